# Initial kernel scaffold; baseline (speedup 1.0000x reference)
import jax
import jax.numpy as jnp
from jax.experimental import pallas as pl


def kernel(x, edge_index, edge_attr, Wn1, Wn2, We1, We2, Wue1, Wue2, Wun1, Wun2, Wd1, Wd2):
    raise NotImplementedError("write your pallas kernel here")


# speedup vs baseline: 2.9339x; 2.9339x over previous
import functools

import jax
import jax.numpy as jnp
from jax import lax
from jax.experimental import pallas as pl
from jax.experimental.pallas import tpu as pltpu
from jax.experimental.pallas import tpu_sc as plsc

LAT = 128
NCORES = 2
NSUB = 16
NW = NCORES * NSUB
CHUNK = 80

@functools.lru_cache(maxsize=1)
def _sc_mesh():
    return plsc.VectorSubcoreMesh(
        core_axis_name="c", subcore_axis_name="s",
        num_cores=NCORES, num_subcores=NSUB)


def _ln(h):
    mu = jnp.mean(h, axis=-1, keepdims=True)
    d = h - mu
    var = jnp.mean(d * d, axis=-1, keepdims=True)
    return d / jnp.sqrt(var + 1e-6)


def _relu(h):
    return jnp.maximum(h, 0.0)


def _node_encode_body(x_ref, wn1, wn2, b, c, nodes_ref, ps_ref, pd_ref):
    h = _relu(_relu(x_ref[...] @ wn1[...]) @ wn2[...])
    nodes = _ln(h)
    nodes_ref[...] = nodes
    ps_ref[...] = nodes @ b[...]
    pd_ref[...] = nodes @ c[...]


def _node_encode(x, wn1, wn2, b, c):
    n = x.shape[0]
    f = jax.ShapeDtypeStruct((n, LAT), jnp.float32)
    return pl.pallas_call(_node_encode_body, out_shape=(f, f, f))(x, wn1, wn2, b, c)


def _node_update_body(nodes_ref, aggs_ref, u1a, u1b, wun2, b, c, ps_ref, pd_ref):
    nodes = nodes_ref[...]
    n = nodes.shape[0]
    agg = aggs_ref[0, :n, :] + aggs_ref[1, :n, :]
    h = _relu(nodes @ u1a[...] + agg @ u1b[...])
    h = _relu(h @ wun2[...])
    n2 = nodes + _ln(h)
    ps_ref[...] = n2 @ b[...]
    pd_ref[...] = n2 @ c[...]


def _node_update(nodes, aggs, u1a, u1b, wun2, b, c):
    n = nodes.shape[0]
    f = jax.ShapeDtypeStruct((n, LAT), jnp.float32)
    return pl.pallas_call(_node_update_body, out_shape=(f, f))(
        nodes, aggs, u1a, u1b, wun2, b, c)


def _edge1_body(ea_ref, g_ref, we1, we2, a, wue2, e1_ref):
    e0 = _ln(_relu(_relu(ea_ref[...] @ we1[...]) @ we2[...]))
    h = _relu(e0 @ a[...] + g_ref[...])
    e1_ref[...] = e0 + _ln(_relu(h @ wue2[...]))


def _edge_step1(ea, g, we1, we2, a, wue2, blk):
    e, de = ea.shape
    grid = (e // blk,)
    full = lambda i: (0, 0)
    return pl.pallas_call(
        _edge1_body,
        grid=grid,
        in_specs=[
            pl.BlockSpec((blk, de), lambda i: (i, 0)),
            pl.BlockSpec((blk, LAT), lambda i: (i, 0)),
            pl.BlockSpec((de, LAT), full),
            pl.BlockSpec((LAT, LAT), full),
            pl.BlockSpec((LAT, LAT), full),
            pl.BlockSpec((LAT, LAT), full),
        ],
        out_specs=pl.BlockSpec((blk, LAT), lambda i: (i, 0)),
        out_shape=jax.ShapeDtypeStruct((e, LAT), jnp.float32),
        compiler_params=pltpu.CompilerParams(
            dimension_semantics=("arbitrary",)),
    )(ea, g, we1, we2, a, wue2)


def _edge2_body(e1_ref, g_ref, a, wue2, wd1, wd2, dec_ref, hp_ref):
    e1 = e1_ref[...]
    h = _relu(e1 @ a[...] + g_ref[...])
    e2 = e1 + _ln(_relu(h @ wue2[...]))
    d = _relu(e2 @ wd1[...]) @ wd2[...]
    dec_ref[...] = d
    hp_ref[...] = jnp.full((1, 1, LAT), jnp.sum(d), dtype=jnp.float32)


def _edge_step2(e1, g, a, wue2, wd1, wd2, blk):
    e = e1.shape[0]
    grid = (e // blk,)
    full = lambda i: (0, 0)
    return pl.pallas_call(
        _edge2_body,
        grid=grid,
        in_specs=[
            pl.BlockSpec((blk, LAT), lambda i: (i, 0)),
            pl.BlockSpec((blk, LAT), lambda i: (i, 0)),
            pl.BlockSpec((LAT, LAT), full),
            pl.BlockSpec((LAT, LAT), full),
            pl.BlockSpec((LAT, LAT), full),
            pl.BlockSpec((LAT, 1), full),
        ],
        out_specs=(
            pl.BlockSpec((blk, 1), lambda i: (i, 0)),
            pl.BlockSpec((1, 1, LAT), lambda i: (i, 0, 0)),
        ),
        out_shape=(
            jax.ShapeDtypeStruct((e, 1), jnp.float32),
            jax.ShapeDtypeStruct((e // blk, 1, LAT), jnp.float32),
        ),
        compiler_params=pltpu.CompilerParams(
            dimension_semantics=("arbitrary",)),
    )(e1, g, a, wue2, wd1, wd2)


def _gather_pair(ps, pd, src3d, dst3d):
    nw, cpw, k = src3d.shape
    e = nw * cpw * k

    @functools.partial(
        pl.kernel, mesh=_sc_mesh(),
        out_type=jax.ShapeDtypeStruct((e, LAT), jnp.float32),
        scratch_types=[
            pltpu.VMEM((cpw, k), jnp.int32),
            pltpu.VMEM((cpw, k), jnp.int32),
            pltpu.VMEM((k, LAT), jnp.float32),
            pltpu.VMEM((k, LAT), jnp.float32),
            pltpu.SemaphoreType.DMA,
        ],
    )
    def kern(ps_hbm, pd_hbm, src_hbm, dst_hbm, out_hbm,
             idxs_v, idxd_v, bufp, bufd, sem):
        wid = lax.axis_index("s") * NCORES + lax.axis_index("c")
        pltpu.sync_copy(src_hbm.at[wid], idxs_v)
        pltpu.sync_copy(dst_hbm.at[wid], idxd_v)

        def body(i, carry):
            cp1 = pltpu.async_copy(ps_hbm.at[idxs_v.at[i]], bufp, sem)
            cp2 = pltpu.async_copy(pd_hbm.at[idxd_v.at[i]], bufd, sem)
            cp1.wait()
            cp2.wait()

            def addrow(r, carry2):
                for j in range(LAT // 16):
                    sl = pl.ds(j * 16, 16)
                    bufp[r, sl] = bufp[r, sl] + bufd[r, sl]
                return carry2
            lax.fori_loop(0, k, addrow, 0)
            pltpu.sync_copy(bufp, out_hbm.at[pl.ds((wid * cpw + i) * k, k)])
            return carry
        lax.fori_loop(0, cpw, body, 0)

    return kern(ps, pd, src3d, dst3d)


def _segment_sum(e1, dst3d, n):
    nw, cpw, k = dst3d.shape
    npad = 10240
    rpt = npad // NSUB
    stg = 128

    @functools.partial(
        pl.kernel, mesh=_sc_mesh(),
        out_type=jax.ShapeDtypeStruct((NCORES, npad, LAT), jnp.float32),
        scratch_types=[
            pltpu.VMEM((cpw, k), jnp.int32),
            pltpu.VMEM((k, LAT), jnp.float32),
            pltpu.VMEM((stg, LAT), jnp.float32),
            pltpu.VMEM_SHARED((npad, LAT), jnp.float32),
            pltpu.SemaphoreType.DMA,
        ],
    )
    def kern(e_hbm, dst_hbm, out_hbm, idxd_v, rows_v, stage, acc_sh, sem):
        c = lax.axis_index("c")
        s = lax.axis_index("s")
        wid = s * NCORES + c
        pltpu.sync_copy(dst_hbm.at[wid], idxd_v)

        def zrow(r, carry):
            for j in range(LAT // 16):
                stage[r, pl.ds(j * 16, 16)] = jnp.zeros((16,), jnp.float32)
            return carry
        lax.fori_loop(0, stg, zrow, 0)
        for t in range(rpt // stg):
            pltpu.sync_copy(stage, acc_sh.at[pl.ds(s * rpt + t * stg, stg)])
        plsc.subcore_barrier()

        def body(i, carry):
            pltpu.sync_copy(e_hbm.at[pl.ds((wid * cpw + i) * k, k)], rows_v)
            pltpu.sync_copy(rows_v, acc_sh.at[idxd_v.at[i]], add=True)
            return carry
        lax.fori_loop(0, cpw, body, 0)
        plsc.subcore_barrier()

        for t in range(rpt // stg):
            sl = pl.ds(s * rpt + t * stg, stg)
            pltpu.sync_copy(acc_sh.at[sl], stage)
            pltpu.sync_copy(stage, out_hbm.at[c, sl])

    return kern(e1, dst3d)


def kernel(x, edge_index, edge_attr, Wn1, Wn2, We1, We2,
           Wue1, Wue2, Wun1, Wun2, Wd1, Wd2):
    n = x.shape[0]
    e = edge_attr.shape[0]
    blk = 2000

    cpw = e // (NW * CHUNK)
    src3d = edge_index[0].astype(jnp.int32).reshape(NW, cpw, CHUNK)
    dst3d = edge_index[1].astype(jnp.int32).reshape(NW, cpw, CHUNK)
    a, b, c = Wue1[:LAT], Wue1[LAT:2 * LAT], Wue1[2 * LAT:]
    u1a, u1b = Wun1[:LAT], Wun1[LAT:]

    nodes, ps, pd = _node_encode(x, Wn1, Wn2, b, c)
    g1 = _gather_pair(ps, pd, src3d, dst3d)
    e1 = _edge_step1(edge_attr, g1, We1, We2, a, Wue2, blk)
    aggs = _segment_sum(e1, dst3d, n)
    ps2, pd2 = _node_update(nodes, aggs, u1a, u1b, Wun2, b, c)
    g2 = _gather_pair(ps2, pd2, src3d, dst3d)
    dec, hpart = _edge_step2(e1, g2, a, Wue2, Wd1, Wd2, blk)
    return dec, jnp.sum(hpart[:, 0, 0])


# speedup vs baseline: 3.6836x; 1.2555x over previous
import functools

import jax
import jax.numpy as jnp
from jax import lax
from jax.experimental import pallas as pl
from jax.experimental.pallas import tpu as pltpu
from jax.experimental.pallas import tpu_sc as plsc

LAT = 128
NCORES = 2
NSUB = 16
NW = NCORES * NSUB
CHUNK = 80

@functools.lru_cache(maxsize=1)
def _sc_mesh():
    return plsc.VectorSubcoreMesh(
        core_axis_name="c", subcore_axis_name="s",
        num_cores=NCORES, num_subcores=NSUB)


def _ln(h):
    mu = jnp.mean(h, axis=-1, keepdims=True)
    d = h - mu
    var = jnp.mean(d * d, axis=-1, keepdims=True)
    return d / jnp.sqrt(var + 1e-6)


def _relu(h):
    return jnp.maximum(h, 0.0)


def _node_encode_body(x_ref, wn1, wn2, b, c, nodes_ref, ps_ref, pd_ref):
    h = _relu(_relu(x_ref[...] @ wn1[...]) @ wn2[...])
    nodes = _ln(h)
    nodes_ref[...] = nodes
    ps_ref[...] = nodes @ b[...]
    pd_ref[...] = nodes @ c[...]


def _node_encode(x, wn1, wn2, b, c):
    n = x.shape[0]
    f = jax.ShapeDtypeStruct((n, LAT), jnp.float32)
    return pl.pallas_call(_node_encode_body, out_shape=(f, f, f))(x, wn1, wn2, b, c)


def _node_update_body(nodes_ref, aggs_ref, u1a, u1b, wun2, b, c, ps_ref, pd_ref):
    nodes = nodes_ref[...]
    n = nodes.shape[0]
    agg = aggs_ref[0, :n, :] + aggs_ref[1, :n, :]
    h = _relu(nodes @ u1a[...] + agg @ u1b[...])
    h = _relu(h @ wun2[...])
    n2 = nodes + _ln(h)
    ps_ref[...] = n2 @ b[...]
    pd_ref[...] = n2 @ c[...]


def _node_update(nodes, aggs, u1a, u1b, wun2, b, c):
    n = nodes.shape[0]
    f = jax.ShapeDtypeStruct((n, LAT), jnp.float32)
    return pl.pallas_call(_node_update_body, out_shape=(f, f))(
        nodes, aggs, u1a, u1b, wun2, b, c)


def _edge1_body(ea_ref, g_ref, we1, we2, a, wue2, e1_ref):
    e0 = _ln(_relu(_relu(ea_ref[...] @ we1[...]) @ we2[...]))
    h = _relu(e0 @ a[...] + g_ref[...])
    e1_ref[...] = e0 + _ln(_relu(h @ wue2[...]))


def _edge_step1(ea, g, we1, we2, a, wue2, blk):
    e, de = ea.shape
    grid = (e // blk,)
    full = lambda i: (0, 0)
    return pl.pallas_call(
        _edge1_body,
        grid=grid,
        in_specs=[
            pl.BlockSpec((blk, de), lambda i: (i, 0)),
            pl.BlockSpec((blk, LAT), lambda i: (i, 0)),
            pl.BlockSpec((de, LAT), full),
            pl.BlockSpec((LAT, LAT), full),
            pl.BlockSpec((LAT, LAT), full),
            pl.BlockSpec((LAT, LAT), full),
        ],
        out_specs=pl.BlockSpec((blk, LAT), lambda i: (i, 0)),
        out_shape=jax.ShapeDtypeStruct((e, LAT), jnp.float32),
        compiler_params=pltpu.CompilerParams(
            dimension_semantics=("arbitrary",)),
    )(ea, g, we1, we2, a, wue2)


def _edge2_body(e1_ref, g_ref, a, wue2, wd1, wd2, dec_ref, hp_ref):
    e1 = e1_ref[...]
    h = _relu(e1 @ a[...] + g_ref[...])
    e2 = e1 + _ln(_relu(h @ wue2[...]))
    d = _relu(e2 @ wd1[...]) @ wd2[...]
    dec_ref[...] = d
    hp_ref[...] = jnp.full((1, 1, LAT), jnp.sum(d), dtype=jnp.float32)


def _edge_step2(e1, g, a, wue2, wd1, wd2, blk):
    e = e1.shape[0]
    grid = (e // blk,)
    full = lambda i: (0, 0)
    return pl.pallas_call(
        _edge2_body,
        grid=grid,
        in_specs=[
            pl.BlockSpec((blk, LAT), lambda i: (i, 0)),
            pl.BlockSpec((blk, LAT), lambda i: (i, 0)),
            pl.BlockSpec((LAT, LAT), full),
            pl.BlockSpec((LAT, LAT), full),
            pl.BlockSpec((LAT, LAT), full),
            pl.BlockSpec((LAT, 1), full),
        ],
        out_specs=(
            pl.BlockSpec((blk, 1), lambda i: (i, 0)),
            pl.BlockSpec((1, 1, LAT), lambda i: (i, 0, 0)),
        ),
        out_shape=(
            jax.ShapeDtypeStruct((e, 1), jnp.float32),
            jax.ShapeDtypeStruct((e // blk, 1, LAT), jnp.float32),
        ),
        compiler_params=pltpu.CompilerParams(
            dimension_semantics=("arbitrary",)),
    )(e1, g, a, wue2, wd1, wd2)


K2 = 128


def _gather_pair(ps, pd, src2, dst2):
    nw, ew = src2.shape
    e = nw * ew
    nch = ew // K2
    tail = ew - nch * K2

    @functools.partial(
        pl.kernel, mesh=_sc_mesh(),
        out_type=jax.ShapeDtypeStruct((e, LAT), jnp.float32),
        scratch_types=[
            pltpu.VMEM((ew,), jnp.int32),
            pltpu.VMEM((ew,), jnp.int32),
            pltpu.VMEM((2, K2, LAT), jnp.float32),
            pltpu.VMEM((2, K2, LAT), jnp.float32),
            pltpu.SemaphoreType.DMA,
            pltpu.SemaphoreType.DMA,
            pltpu.SemaphoreType.DMA,
            pltpu.SemaphoreType.DMA,
        ],
    )
    def kern(ps_hbm, pd_hbm, src_hbm, dst_hbm, out_hbm,
             idxs_v, idxd_v, bufp, bufd, g0, g1, w0, w1):
        wid = lax.axis_index("s") * NCORES + lax.axis_index("c")
        wbase = pl.multiple_of(wid * ew, 8)
        pltpu.sync_copy(src_hbm.at[wid], idxs_v)
        pltpu.sync_copy(dst_hbm.at[wid], idxd_v)
        gsem = (g0, g1)
        wsem = (w0, w1)

        def isl(v, i, k=K2):
            return v.at[pl.ds(pl.multiple_of(i * K2, 8), k)]

        def orows(i, k=K2):
            return out_hbm.at[pl.ds(pl.multiple_of(wbase + i * K2, 8), k)]

        def start_gather(i, b):
            pltpu.async_copy(ps_hbm.at[isl(idxs_v, i)], bufp.at[b], gsem[b])
            pltpu.async_copy(pd_hbm.at[isl(idxd_v, i)], bufd.at[b], gsem[b])

        def wait_gather(i, b):
            pltpu.make_async_copy(ps_hbm.at[isl(idxs_v, i)], bufp.at[b], gsem[b]).wait()
            pltpu.make_async_copy(pd_hbm.at[isl(idxd_v, i)], bufd.at[b], gsem[b]).wait()

        def add_buf(b, k):
            def addrow(r, carry):
                for j in range(LAT // 16):
                    sl = pl.ds(j * 16, 16)
                    bufp[b, r, sl] = bufp[b, r, sl] + bufd[b, r, sl]
                return carry
            lax.fori_loop(0, k, addrow, 0)

        start_gather(0, 0)
        start_gather(1, 1)

        def body(g, carry):
            for b in range(2):
                i = 2 * g + b
                wait_gather(i, b)
                add_buf(b, K2)
                pltpu.async_copy(bufp.at[b], orows(i), wsem[b])
            for b in range(2):
                i = 2 * g + b

                @pl.when(i + 2 < nch)
                def _():
                    pltpu.make_async_copy(bufp.at[b], orows(i), wsem[b]).wait()
                    start_gather(i + 2, b)
            return carry
        lax.fori_loop(0, nch // 2, body, 0)
        for b in range(2):
            pltpu.make_async_copy(bufp.at[b], orows(nch - 2 + b), wsem[b]).wait()
        if tail:
            cps = pltpu.async_copy(
                ps_hbm.at[isl(idxs_v, nch, tail)], bufp.at[0].at[pl.ds(0, tail)], g0)
            cpd = pltpu.async_copy(
                pd_hbm.at[isl(idxd_v, nch, tail)], bufd.at[0].at[pl.ds(0, tail)], g0)
            cps.wait()
            cpd.wait()
            add_buf(0, tail)
            pltpu.sync_copy(bufp.at[0].at[pl.ds(0, tail)], orows(nch, tail))

    return kern(ps, pd, src2, dst2)


def _segment_sum(e1, dstp, n):
    nw, nchp, ks = dstp.shape
    nch = nchp - 1
    ew = n * 32 // NW
    tail = ew - nch * ks
    npad = 10240
    rpt = npad // NSUB
    stg = 64

    @functools.partial(
        pl.kernel, mesh=_sc_mesh(),
        out_type=jax.ShapeDtypeStruct((NCORES, npad, LAT), jnp.float32),
        scratch_types=[
            pltpu.VMEM((nchp, ks), jnp.int32),
            pltpu.VMEM((2, ks, LAT), jnp.float32),
            pltpu.VMEM((stg, LAT), jnp.float32),
            pltpu.VMEM_SHARED((npad, LAT), jnp.float32),
            pltpu.SemaphoreType.DMA,
            pltpu.SemaphoreType.DMA,
        ],
    )
    def kern(e_hbm, dst_hbm, out_hbm, idxd_v, rows, stage, acc_sh, r0, r1):
        c = lax.axis_index("c")
        s = lax.axis_index("s")
        wid = s * NCORES + c
        wbase = pl.multiple_of(wid * ew, 8)
        rsem = (r0, r1)

        def erows(i, k=ks):
            return e_hbm.at[pl.ds(pl.multiple_of(wbase + i * ks, 8), k)]

        pltpu.sync_copy(dst_hbm.at[wid], idxd_v)
        pltpu.async_copy(erows(0), rows.at[0], r0)
        pltpu.async_copy(erows(1), rows.at[1], r1)

        def zrows(ref, lo, cnt):
            def zrow(r, carry):
                for j in range(LAT // 16):
                    ref[r, pl.ds(j * 16, 16)] = jnp.zeros((16,), jnp.float32)
                return carry
            lax.fori_loop(lo, lo + cnt, zrow, 0)

        zrows(stage, 0, stg)
        for t in range(rpt // stg):
            pltpu.sync_copy(
                stage, acc_sh.at[pl.ds(pl.multiple_of(s * rpt + t * stg, 8), stg)])
        plsc.subcore_barrier()

        def body(g, carry):
            for b in range(2):
                i = 2 * g + b
                pltpu.make_async_copy(erows(i), rows.at[b], rsem[b]).wait()
                pltpu.sync_copy(rows.at[b], acc_sh.at[idxd_v.at[i]], add=True)

                @pl.when(i + 2 < nch)
                def _():
                    pltpu.async_copy(erows(i + 2), rows.at[b], rsem[b])
            return carry
        lax.fori_loop(0, nch // 2, body, 0)
        pltpu.async_copy(erows(nch, tail), rows.at[0].at[pl.ds(0, tail)], r0).wait()
        def zrow2(r, carry):
            for j in range(LAT // 16):
                rows[0, r, pl.ds(j * 16, 16)] = jnp.zeros((16,), jnp.float32)
            return carry
        lax.fori_loop(tail, ks, zrow2, 0)
        pltpu.sync_copy(rows.at[0], acc_sh.at[idxd_v.at[nch]], add=True)
        plsc.subcore_barrier()

        for t in range(rpt // stg):
            sl = pl.ds(pl.multiple_of(s * rpt + t * stg, 8), stg)
            pltpu.sync_copy(acc_sh.at[sl], stage)
            pltpu.sync_copy(stage, out_hbm.at[c].at[sl])

    return kern(e1, dstp)


def kernel(x, edge_index, edge_attr, Wn1, Wn2, We1, We2,
           Wue1, Wue2, Wun1, Wun2, Wd1, Wd2):
    n = x.shape[0]
    e = edge_attr.shape[0]
    blk = 2000
    ew = e // NW

    src2 = edge_index[0].astype(jnp.int32).reshape(NW, ew)
    dst2 = edge_index[1].astype(jnp.int32).reshape(NW, ew)
    ks = 64
    nchs = ew // ks
    pad = (nchs + 1) * ks - ew
    dstp = jnp.concatenate(
        [dst2, jnp.full((NW, pad), 10239, jnp.int32)], axis=1
    ).reshape(NW, nchs + 1, ks)
    a, b, c = Wue1[:LAT], Wue1[LAT:2 * LAT], Wue1[2 * LAT:]
    u1a, u1b = Wun1[:LAT], Wun1[LAT:]

    nodes, ps, pd = _node_encode(x, Wn1, Wn2, b, c)
    g1 = _gather_pair(ps, pd, src2, dst2)
    e1 = _edge_step1(edge_attr, g1, We1, We2, a, Wue2, blk)
    aggs = _segment_sum(e1, dstp, n)
    ps2, pd2 = _node_update(nodes, aggs, u1a, u1b, Wun2, b, c)
    g2 = _gather_pair(ps2, pd2, src2, dst2)
    dec, hpart = _edge_step2(e1, g2, a, Wue2, Wd1, Wd2, blk)
    return dec, jnp.sum(hpart[:, 0, 0])


# speedup vs baseline: 4.0135x; 1.0896x over previous
import functools

import jax
import jax.numpy as jnp
from jax import lax
from jax.experimental import pallas as pl
from jax.experimental.pallas import tpu as pltpu
from jax.experimental.pallas import tpu_sc as plsc

LAT = 128
NCORES = 2
NSUB = 16
NW = NCORES * NSUB
CHUNK = 80

@functools.lru_cache(maxsize=1)
def _sc_mesh():
    return plsc.VectorSubcoreMesh(
        core_axis_name="c", subcore_axis_name="s",
        num_cores=NCORES, num_subcores=NSUB)


def _ln(h):
    mu = jnp.mean(h, axis=-1, keepdims=True)
    d = h - mu
    var = jnp.mean(d * d, axis=-1, keepdims=True)
    return d / jnp.sqrt(var + 1e-6)


def _relu(h):
    return jnp.maximum(h, 0.0)


def _node_encode_body(x_ref, wn1, wn2, b, c, nodes_ref, ps_ref, pd_ref):
    h = _relu(_relu(x_ref[...] @ wn1[...]) @ wn2[...])
    nodes = _ln(h)
    nodes_ref[...] = nodes
    ps_ref[...] = nodes @ b[...]
    pd_ref[...] = nodes @ c[...]


def _node_encode(x, wn1, wn2, b, c):
    n = x.shape[0]
    f = jax.ShapeDtypeStruct((n, LAT), jnp.float32)
    return pl.pallas_call(_node_encode_body, out_shape=(f, f, f))(x, wn1, wn2, b, c)


def _node_update_body(nodes_ref, *rest):
    aggs_refs = rest[:-7]
    u1a, u1b, wun2, b, c, ps_ref, pd_ref = rest[-7:]
    nodes = nodes_ref[...]
    n = nodes.shape[0]
    agg = sum(ar[i, :n, :] for ar in aggs_refs for i in range(NCORES))
    h = _relu(nodes @ u1a[...] + agg @ u1b[...])
    h = _relu(h @ wun2[...])
    n2 = nodes + _ln(h)
    ps_ref[...] = n2 @ b[...]
    pd_ref[...] = n2 @ c[...]


def _node_update(nodes, aggs, u1a, u1b, wun2, b, c):
    n = nodes.shape[0]
    f = jax.ShapeDtypeStruct((n, LAT), jnp.float32)
    return pl.pallas_call(_node_update_body, out_shape=(f, f))(
        nodes, *aggs, u1a, u1b, wun2, b, c)


def _edge1_body(ea_ref, g_ref, we1, we2, a, wue2, e1_ref):
    e0 = _ln(_relu(_relu(ea_ref[...] @ we1[...]) @ we2[...]))
    h = _relu(e0 @ a[...] + g_ref[...])
    e1_ref[...] = e0 + _ln(_relu(h @ wue2[...]))


def _edge_step1(ea, g, we1, we2, a, wue2, blk):
    e, de = ea.shape
    grid = (e // blk,)
    full = lambda i: (0, 0)
    return pl.pallas_call(
        _edge1_body,
        grid=grid,
        in_specs=[
            pl.BlockSpec((blk, de), lambda i: (i, 0)),
            pl.BlockSpec((blk, LAT), lambda i: (i, 0)),
            pl.BlockSpec((de, LAT), full),
            pl.BlockSpec((LAT, LAT), full),
            pl.BlockSpec((LAT, LAT), full),
            pl.BlockSpec((LAT, LAT), full),
        ],
        out_specs=pl.BlockSpec((blk, LAT), lambda i: (i, 0)),
        out_shape=jax.ShapeDtypeStruct((e, LAT), jnp.float32),
        compiler_params=pltpu.CompilerParams(
            dimension_semantics=("arbitrary",)),
    )(ea, g, we1, we2, a, wue2)


def _edge2_body(e1_ref, g_ref, a, wue2, wd1, wd2, dec_ref, hp_ref):
    e1 = e1_ref[...]
    h = _relu(e1 @ a[...] + g_ref[...])
    e2 = e1 + _ln(_relu(h @ wue2[...]))
    d = _relu(e2 @ wd1[...]) @ wd2[...]
    dec_ref[...] = d
    hp_ref[...] = jnp.full((1, 1, LAT), jnp.sum(d), dtype=jnp.float32)


def _edge_step2(e1, g, a, wue2, wd1, wd2, blk):
    e = e1.shape[0]
    grid = (e // blk,)
    full = lambda i: (0, 0)
    return pl.pallas_call(
        _edge2_body,
        grid=grid,
        in_specs=[
            pl.BlockSpec((blk, LAT), lambda i: (i, 0)),
            pl.BlockSpec((blk, LAT), lambda i: (i, 0)),
            pl.BlockSpec((LAT, LAT), full),
            pl.BlockSpec((LAT, LAT), full),
            pl.BlockSpec((LAT, LAT), full),
            pl.BlockSpec((LAT, 1), full),
        ],
        out_specs=(
            pl.BlockSpec((blk, 1), lambda i: (i, 0)),
            pl.BlockSpec((1, 1, LAT), lambda i: (i, 0, 0)),
        ),
        out_shape=(
            jax.ShapeDtypeStruct((e, 1), jnp.float32),
            jax.ShapeDtypeStruct((e // blk, 1, LAT), jnp.float32),
        ),
        compiler_params=pltpu.CompilerParams(
            dimension_semantics=("arbitrary",)),
    )(e1, g, a, wue2, wd1, wd2)


K2 = 128


def _gather_pair(ps, pd, src2, dst2):
    nw, ew = src2.shape
    e = nw * ew
    nch = ew // K2
    nchr = (nch // 2) * 2
    tail = ew - nch * K2

    @functools.partial(
        pl.kernel, mesh=_sc_mesh(),
        out_type=jax.ShapeDtypeStruct((e, LAT), jnp.float32),
        scratch_types=[
            pltpu.VMEM((ew,), jnp.int32),
            pltpu.VMEM((ew,), jnp.int32),
            pltpu.VMEM((2, K2, LAT), jnp.float32),
            pltpu.VMEM((2, K2, LAT), jnp.float32),
            pltpu.SemaphoreType.DMA,
            pltpu.SemaphoreType.DMA,
            pltpu.SemaphoreType.DMA,
            pltpu.SemaphoreType.DMA,
        ],
    )
    def kern(ps_hbm, pd_hbm, src_hbm, dst_hbm, out_hbm,
             idxs_v, idxd_v, bufp, bufd, g0, g1, w0, w1):
        wid = lax.axis_index("s") * NCORES + lax.axis_index("c")
        wbase = pl.multiple_of(wid * ew, 8)
        pltpu.sync_copy(src_hbm.at[wid], idxs_v)
        pltpu.sync_copy(dst_hbm.at[wid], idxd_v)
        gsem = (g0, g1)
        wsem = (w0, w1)

        def isl(v, i, k=K2):
            return v.at[pl.ds(pl.multiple_of(i * K2, 8), k)]

        def orows(i, k=K2):
            return out_hbm.at[pl.ds(pl.multiple_of(wbase + i * K2, 8), k)]

        def start_gather(i, b):
            pltpu.async_copy(ps_hbm.at[isl(idxs_v, i)], bufp.at[b], gsem[b])
            pltpu.async_copy(pd_hbm.at[isl(idxd_v, i)], bufd.at[b], gsem[b])

        def wait_gather(i, b):
            pltpu.make_async_copy(ps_hbm.at[isl(idxs_v, i)], bufp.at[b], gsem[b]).wait()
            pltpu.make_async_copy(pd_hbm.at[isl(idxd_v, i)], bufd.at[b], gsem[b]).wait()

        def add_buf(b, k):
            def addrow(r, carry):
                for j in range(LAT // 16):
                    sl = pl.ds(j * 16, 16)
                    bufp[b, r, sl] = bufp[b, r, sl] + bufd[b, r, sl]
                return carry
            lax.fori_loop(0, k, addrow, 0)

        start_gather(0, 0)
        start_gather(1, 1)

        def body(g, carry):
            for b in range(2):
                i = 2 * g + b
                wait_gather(i, b)
                add_buf(b, K2)
                pltpu.async_copy(bufp.at[b], orows(i), wsem[b])
            for b in range(2):
                i = 2 * g + b

                @pl.when(i + 2 < nchr)
                def _():
                    pltpu.make_async_copy(bufp.at[b], orows(i), wsem[b]).wait()
                    start_gather(i + 2, b)
            return carry
        lax.fori_loop(0, nchr // 2, body, 0)
        for b in range(2):
            pltpu.make_async_copy(bufp.at[b], orows(nchr - 2 + b), wsem[b]).wait()
        if nch > nchr:
            start_gather(nchr, 0)
            wait_gather(nchr, 0)
            add_buf(0, K2)
            pltpu.sync_copy(bufp.at[0], orows(nchr))
        if tail:
            cps = pltpu.async_copy(
                ps_hbm.at[isl(idxs_v, nch, tail)], bufp.at[0].at[pl.ds(0, tail)], g0)
            cpd = pltpu.async_copy(
                pd_hbm.at[isl(idxd_v, nch, tail)], bufd.at[0].at[pl.ds(0, tail)], g0)
            cps.wait()
            cpd.wait()
            add_buf(0, tail)
            pltpu.sync_copy(bufp.at[0].at[pl.ds(0, tail)], orows(nch, tail))

    return kern(ps, pd, src2, dst2)


def _segment_sum(e1, dstp, ew):
    nw, nchp, ks = dstp.shape
    nch = nchp - 1
    nchr = (nch // 2) * 2
    tail = ew - nch * ks
    npad = NPAD
    rpt = npad // NSUB
    stg = 64

    @functools.partial(
        pl.kernel, mesh=_sc_mesh(),
        out_type=jax.ShapeDtypeStruct((NCORES, npad, LAT), jnp.float32),
        scratch_types=[
            pltpu.VMEM((nchp, ks), jnp.int32),
            pltpu.VMEM((2, ks, LAT), jnp.float32),
            pltpu.VMEM((stg, LAT), jnp.float32),
            pltpu.VMEM_SHARED((npad, LAT), jnp.float32),
            pltpu.SemaphoreType.DMA,
            pltpu.SemaphoreType.DMA,
        ],
    )
    def kern(e_hbm, dst_hbm, out_hbm, idxd_v, rows, stage, acc_sh, r0, r1):
        c = lax.axis_index("c")
        s = lax.axis_index("s")
        wid = s * NCORES + c
        wbase = pl.multiple_of(wid * ew, 8)
        rsem = (r0, r1)

        def erows(i, k=ks):
            return e_hbm.at[pl.ds(pl.multiple_of(wbase + i * ks, 8), k)]

        pltpu.sync_copy(dst_hbm.at[wid], idxd_v)
        pltpu.async_copy(erows(0), rows.at[0], r0)
        pltpu.async_copy(erows(1), rows.at[1], r1)

        def zrows(ref, lo, cnt):
            def zrow(r, carry):
                for j in range(LAT // 16):
                    ref[r, pl.ds(j * 16, 16)] = jnp.zeros((16,), jnp.float32)
                return carry
            lax.fori_loop(lo, lo + cnt, zrow, 0)

        zrows(stage, 0, stg)
        for t in range(rpt // stg):
            pltpu.sync_copy(
                stage, acc_sh.at[pl.ds(pl.multiple_of(s * rpt + t * stg, 8), stg)])
        plsc.subcore_barrier()

        def body(g, carry):
            for b in range(2):
                i = 2 * g + b
                pltpu.make_async_copy(erows(i), rows.at[b], rsem[b]).wait()
                pltpu.sync_copy(rows.at[b], acc_sh.at[idxd_v.at[i]], add=True)

                @pl.when(i + 2 < nchr)
                def _():
                    pltpu.async_copy(erows(i + 2), rows.at[b], rsem[b])
            return carry
        lax.fori_loop(0, nchr // 2, body, 0)
        if nch > nchr:
            pltpu.async_copy(erows(nchr), rows.at[0], r0).wait()
            pltpu.sync_copy(rows.at[0], acc_sh.at[idxd_v.at[nchr]], add=True)
        pltpu.async_copy(erows(nch, tail), rows.at[0].at[pl.ds(0, tail)], r0).wait()
        def zrow2(r, carry):
            for j in range(LAT // 16):
                rows[0, r, pl.ds(j * 16, 16)] = jnp.zeros((16,), jnp.float32)
            return carry
        lax.fori_loop(tail, ks, zrow2, 0)
        pltpu.sync_copy(rows.at[0], acc_sh.at[idxd_v.at[nch]], add=True)
        plsc.subcore_barrier()

        for t in range(rpt // stg):
            sl = pl.ds(pl.multiple_of(s * rpt + t * stg, 8), stg)
            pltpu.sync_copy(acc_sh.at[sl], stage)
            pltpu.sync_copy(stage, out_hbm.at[c].at[sl])

    return kern(e1, dstp)


NSLAB = 2
NPAD = 10240


def kernel(x, edge_index, edge_attr, Wn1, Wn2, We1, We2,
           Wue1, Wue2, Wun1, Wun2, Wd1, Wd2):
    n = x.shape[0]
    e = edge_attr.shape[0]
    blk = 2000
    es = e // NSLAB
    ews = es // NW

    src = edge_index[0].astype(jnp.int32).reshape(NSLAB, NW, ews)
    dst = edge_index[1].astype(jnp.int32).reshape(NSLAB, NW, ews)
    ks = 64
    nchs = ews // ks
    pad = (nchs + 1) * ks - ews
    dstp = jnp.concatenate(
        [dst, jnp.full((NSLAB, NW, pad), NPAD - 1, jnp.int32)], axis=2
    ).reshape(NSLAB, NW, nchs + 1, ks)
    a, b, c = Wue1[:LAT], Wue1[LAT:2 * LAT], Wue1[2 * LAT:]
    u1a, u1b = Wun1[:LAT], Wun1[LAT:]

    nodes, ps, pd = _node_encode(x, Wn1, Wn2, b, c)
    g1 = [_gather_pair(ps, pd, src[t], dst[t]) for t in range(NSLAB)]
    e1, aggs = [], []
    for t in range(NSLAB):
        ea_t = lax.slice_in_dim(edge_attr, t * es, (t + 1) * es)
        e1.append(_edge_step1(ea_t, g1[t], We1, We2, a, Wue2, blk))
        aggs.append(_segment_sum(e1[t], dstp[t], ews))
    ps2, pd2 = _node_update(nodes, aggs, u1a, u1b, Wun2, b, c)
    dec, hs = [], []
    for t in range(NSLAB):
        g2_t = _gather_pair(ps2, pd2, src[t], dst[t])
        dec_t, hp_t = _edge_step2(e1[t], g2_t, a, Wue2, Wd1, Wd2, blk)
        dec.append(dec_t)
        hs.append(jnp.sum(hp_t[:, 0, 0]))
    return jnp.concatenate(dec, axis=0), sum(hs)


# speedup vs baseline: 5.0387x; 1.2554x over previous
import functools

import jax
import jax.numpy as jnp
from jax import lax
from jax.experimental import pallas as pl
from jax.experimental.pallas import tpu as pltpu
from jax.experimental.pallas import tpu_sc as plsc

LAT = 128
NCORES = 2
NSUB = 16
NW = NCORES * NSUB
CHUNK = 80

@functools.lru_cache(maxsize=1)
def _sc_mesh():
    return plsc.VectorSubcoreMesh(
        core_axis_name="c", subcore_axis_name="s",
        num_cores=NCORES, num_subcores=NSUB)


def _ln(h):
    mu = jnp.mean(h, axis=-1, keepdims=True)
    d = h - mu
    var = jnp.mean(d * d, axis=-1, keepdims=True)
    return d / jnp.sqrt(var + 1e-6)


def _relu(h):
    return jnp.maximum(h, 0.0)


def _node_encode_body(x_ref, wn1, wn2, b, c, nodes_ref, ps_ref, pd_ref):
    h = _relu(_relu(x_ref[...] @ wn1[...]) @ wn2[...])
    nodes = _ln(h)
    nodes_ref[...] = nodes
    ps_ref[...] = nodes @ b[...]
    pd_ref[...] = nodes @ c[...]


def _node_encode(x, wn1, wn2, b, c):
    n = x.shape[0]
    f = jax.ShapeDtypeStruct((n, LAT), jnp.float32)
    return pl.pallas_call(_node_encode_body, out_shape=(f, f, f))(x, wn1, wn2, b, c)


def _node_update_body(nodes_ref, *rest):
    aggs_refs = rest[:-7]
    u1a, u1b, wun2, b, c, ps_ref, pd_ref = rest[-7:]
    nodes = nodes_ref[...]
    n = nodes.shape[0]
    agg = sum(ar[i, :n, :] for ar in aggs_refs for i in range(NCORES))
    h = _relu(nodes @ u1a[...] + agg @ u1b[...])
    h = _relu(h @ wun2[...])
    n2 = nodes + _ln(h)
    ps_ref[...] = n2 @ b[...]
    pd_ref[...] = n2 @ c[...]


def _node_update(nodes, aggs, u1a, u1b, wun2, b, c):
    n = nodes.shape[0]
    f = jax.ShapeDtypeStruct((n, LAT), jnp.float32)
    return pl.pallas_call(_node_update_body, out_shape=(f, f))(
        nodes, *aggs, u1a, u1b, wun2, b, c)


def _edge1_body(ea_ref, g_ref, we1, we2, a, wue2, e1_ref):
    h0 = lax.dot_general(ea_ref[...], we1[...], (((0,), (0,)), ((), ())))
    e0 = _ln(_relu(_relu(h0) @ we2[...]))
    h = _relu(e0 @ a[...] + g_ref[...])
    e1_ref[...] = e0 + _ln(_relu(h @ wue2[...]))


def _edge_step1(ea_t, g, we1, we2, a, wue2, blk):
    de, e = ea_t.shape
    grid = (e // blk,)
    full = lambda i: (0, 0)
    return pl.pallas_call(
        _edge1_body,
        grid=grid,
        in_specs=[
            pl.BlockSpec((de, blk), lambda i: (0, i)),
            pl.BlockSpec((blk, LAT), lambda i: (i, 0)),
            pl.BlockSpec((de, LAT), full),
            pl.BlockSpec((LAT, LAT), full),
            pl.BlockSpec((LAT, LAT), full),
            pl.BlockSpec((LAT, LAT), full),
        ],
        out_specs=pl.BlockSpec((blk, LAT), lambda i: (i, 0)),
        out_shape=jax.ShapeDtypeStruct((e, LAT), jnp.float32),
        compiler_params=pltpu.CompilerParams(
            dimension_semantics=("arbitrary",)),
    )(ea_t, g, we1, we2, a, wue2)


def _edge2_body(e1_ref, g_ref, a, wue2, wd1, wd2, dec_ref, hp_ref):
    e1 = e1_ref[...]
    h = _relu(e1 @ a[...] + g_ref[...])
    e2 = e1 + _ln(_relu(h @ wue2[...]))
    h2 = _relu(e2 @ wd1[...])
    d = lax.dot_general(wd2[...], h2, (((0,), (1,)), ((), ())))
    dec_ref[...] = d
    hp_ref[...] = jnp.full((1, 1, LAT), jnp.sum(d), dtype=jnp.float32)


def _edge_step2(e1, g, a, wue2, wd1, wd2, blk):
    e = e1.shape[0]
    grid = (e // blk,)
    full = lambda i: (0, 0)
    return pl.pallas_call(
        _edge2_body,
        grid=grid,
        in_specs=[
            pl.BlockSpec((blk, LAT), lambda i: (i, 0)),
            pl.BlockSpec((blk, LAT), lambda i: (i, 0)),
            pl.BlockSpec((LAT, LAT), full),
            pl.BlockSpec((LAT, LAT), full),
            pl.BlockSpec((LAT, LAT), full),
            pl.BlockSpec((LAT, 1), full),
        ],
        out_specs=(
            pl.BlockSpec((1, blk), lambda i: (0, i)),
            pl.BlockSpec((1, 1, LAT), lambda i: (i, 0, 0)),
        ),
        out_shape=(
            jax.ShapeDtypeStruct((1, e), jnp.float32),
            jax.ShapeDtypeStruct((e // blk, 1, LAT), jnp.float32),
        ),
        compiler_params=pltpu.CompilerParams(
            dimension_semantics=("arbitrary",)),
    )(e1, g, a, wue2, wd1, wd2)


K2 = 128


def _gather_pair(ps, pd, src2, dst2):
    nw, ew = src2.shape
    e = nw * ew
    nch = ew // K2
    nchr = (nch // 2) * 2
    tail = ew - nch * K2

    @functools.partial(
        pl.kernel, mesh=_sc_mesh(),
        out_type=jax.ShapeDtypeStruct((e, LAT), jnp.float32),
        scratch_types=[
            pltpu.VMEM((ew,), jnp.int32),
            pltpu.VMEM((ew,), jnp.int32),
            pltpu.VMEM((2, K2, LAT), jnp.float32),
            pltpu.VMEM((2, K2, LAT), jnp.float32),
            pltpu.SemaphoreType.DMA,
            pltpu.SemaphoreType.DMA,
            pltpu.SemaphoreType.DMA,
            pltpu.SemaphoreType.DMA,
        ],
    )
    def kern(ps_hbm, pd_hbm, src_hbm, dst_hbm, out_hbm,
             idxs_v, idxd_v, bufp, bufd, g0, g1, w0, w1):
        wid = lax.axis_index("s") * NCORES + lax.axis_index("c")
        wbase = pl.multiple_of(wid * ew, 8)
        pltpu.sync_copy(src_hbm.at[wid], idxs_v)
        pltpu.sync_copy(dst_hbm.at[wid], idxd_v)
        gsem = (g0, g1)
        wsem = (w0, w1)

        def isl(v, i, k=K2):
            return v.at[pl.ds(pl.multiple_of(i * K2, 8), k)]

        def orows(i, k=K2):
            return out_hbm.at[pl.ds(pl.multiple_of(wbase + i * K2, 8), k)]

        def start_gather(i, b):
            pltpu.async_copy(ps_hbm.at[isl(idxs_v, i)], bufp.at[b], gsem[b])
            pltpu.async_copy(pd_hbm.at[isl(idxd_v, i)], bufd.at[b], gsem[b])

        def wait_gather(i, b):
            pltpu.make_async_copy(ps_hbm.at[isl(idxs_v, i)], bufp.at[b], gsem[b]).wait()
            pltpu.make_async_copy(pd_hbm.at[isl(idxd_v, i)], bufd.at[b], gsem[b]).wait()

        def add_buf(b, k):
            def addrow(r, carry):
                for j in range(LAT // 16):
                    sl = pl.ds(j * 16, 16)
                    bufp[b, r, sl] = bufp[b, r, sl] + bufd[b, r, sl]
                return carry
            lax.fori_loop(0, k, addrow, 0)

        start_gather(0, 0)
        start_gather(1, 1)

        def body(g, carry):
            for b in range(2):
                i = 2 * g + b
                wait_gather(i, b)
                add_buf(b, K2)
                pltpu.async_copy(bufp.at[b], orows(i), wsem[b])
            for b in range(2):
                i = 2 * g + b

                @pl.when(i + 2 < nchr)
                def _():
                    pltpu.make_async_copy(bufp.at[b], orows(i), wsem[b]).wait()
                    start_gather(i + 2, b)
            return carry
        lax.fori_loop(0, nchr // 2, body, 0)
        for b in range(2):
            pltpu.make_async_copy(bufp.at[b], orows(nchr - 2 + b), wsem[b]).wait()
        if nch > nchr:
            start_gather(nchr, 0)
            wait_gather(nchr, 0)
            add_buf(0, K2)
            pltpu.sync_copy(bufp.at[0], orows(nchr))
        if tail:
            cps = pltpu.async_copy(
                ps_hbm.at[isl(idxs_v, nch, tail)], bufp.at[0].at[pl.ds(0, tail)], g0)
            cpd = pltpu.async_copy(
                pd_hbm.at[isl(idxd_v, nch, tail)], bufd.at[0].at[pl.ds(0, tail)], g0)
            cps.wait()
            cpd.wait()
            add_buf(0, tail)
            pltpu.sync_copy(bufp.at[0].at[pl.ds(0, tail)], orows(nch, tail))

    return kern(ps, pd, src2, dst2)


def _segment_sum(e1, dstp, ew):
    nw, nchp, ks = dstp.shape
    nch = nchp - 1
    nchr = (nch // 2) * 2
    tail = ew - nch * ks
    npad = NPAD
    rpt = npad // NSUB
    stg = 64

    @functools.partial(
        pl.kernel, mesh=_sc_mesh(),
        out_type=jax.ShapeDtypeStruct((NCORES, npad, LAT), jnp.float32),
        scratch_types=[
            pltpu.VMEM((nchp, ks), jnp.int32),
            pltpu.VMEM((2, ks, LAT), jnp.float32),
            pltpu.VMEM((stg, LAT), jnp.float32),
            pltpu.VMEM_SHARED((npad, LAT), jnp.float32),
            pltpu.SemaphoreType.DMA,
            pltpu.SemaphoreType.DMA,
        ],
    )
    def kern(e_hbm, dst_hbm, out_hbm, idxd_v, rows, stage, acc_sh, r0, r1):
        c = lax.axis_index("c")
        s = lax.axis_index("s")
        wid = s * NCORES + c
        wbase = pl.multiple_of(wid * ew, 8)
        rsem = (r0, r1)

        def erows(i, k=ks):
            return e_hbm.at[pl.ds(pl.multiple_of(wbase + i * ks, 8), k)]

        pltpu.sync_copy(dst_hbm.at[wid], idxd_v)
        pltpu.async_copy(erows(0), rows.at[0], r0)
        pltpu.async_copy(erows(1), rows.at[1], r1)

        def zrows(ref, lo, cnt):
            def zrow(r, carry):
                for j in range(LAT // 16):
                    ref[r, pl.ds(j * 16, 16)] = jnp.zeros((16,), jnp.float32)
                return carry
            lax.fori_loop(lo, lo + cnt, zrow, 0)

        zrows(stage, 0, stg)
        for t in range(rpt // stg):
            pltpu.sync_copy(
                stage, acc_sh.at[pl.ds(pl.multiple_of(s * rpt + t * stg, 8), stg)])
        plsc.subcore_barrier()

        def body(g, carry):
            for b in range(2):
                i = 2 * g + b
                pltpu.make_async_copy(erows(i), rows.at[b], rsem[b]).wait()
                pltpu.sync_copy(rows.at[b], acc_sh.at[idxd_v.at[i]], add=True)

                @pl.when(i + 2 < nchr)
                def _():
                    pltpu.async_copy(erows(i + 2), rows.at[b], rsem[b])
            return carry
        lax.fori_loop(0, nchr // 2, body, 0)
        if nch > nchr:
            pltpu.async_copy(erows(nchr), rows.at[0], r0).wait()
            pltpu.sync_copy(rows.at[0], acc_sh.at[idxd_v.at[nchr]], add=True)
        pltpu.async_copy(erows(nch, tail), rows.at[0].at[pl.ds(0, tail)], r0).wait()
        def zrow2(r, carry):
            for j in range(LAT // 16):
                rows[0, r, pl.ds(j * 16, 16)] = jnp.zeros((16,), jnp.float32)
            return carry
        lax.fori_loop(tail, ks, zrow2, 0)
        pltpu.sync_copy(rows.at[0], acc_sh.at[idxd_v.at[nch]], add=True)
        plsc.subcore_barrier()

        for t in range(rpt // stg):
            sl = pl.ds(pl.multiple_of(s * rpt + t * stg, 8), stg)
            pltpu.sync_copy(acc_sh.at[sl], stage)
            pltpu.sync_copy(stage, out_hbm.at[c].at[sl])

    return kern(e1, dstp)


NSLAB = 2
NPAD = 10240


def kernel(x, edge_index, edge_attr, Wn1, Wn2, We1, We2,
           Wue1, Wue2, Wun1, Wun2, Wd1, Wd2):
    n = x.shape[0]
    e = edge_attr.shape[0]
    blk = 3200
    es = e // NSLAB
    ews = es // NW

    src = edge_index[0].astype(jnp.int32).reshape(NSLAB, NW, ews)
    dst = edge_index[1].astype(jnp.int32).reshape(NSLAB, NW, ews)
    ks = 64
    nchs = ews // ks
    pad = (nchs + 1) * ks - ews
    dstp = jnp.concatenate(
        [dst, jnp.full((NSLAB, NW, pad), NPAD - 1, jnp.int32)], axis=2
    ).reshape(NSLAB, NW, nchs + 1, ks)
    a, b, c = Wue1[:LAT], Wue1[LAT:2 * LAT], Wue1[2 * LAT:]
    u1a, u1b = Wun1[:LAT], Wun1[LAT:]

    nodes, ps, pd = _node_encode(x, Wn1, Wn2, b, c)
    g1 = [_gather_pair(ps, pd, src[t], dst[t]) for t in range(NSLAB)]
    e1, aggs = [], []
    for t in range(NSLAB):
        ea_t = lax.slice_in_dim(edge_attr.T, t * es, (t + 1) * es, axis=1)
        e1.append(_edge_step1(ea_t, g1[t], We1, We2, a, Wue2, blk))
        aggs.append(_segment_sum(e1[t], dstp[t], ews))
    ps2, pd2 = _node_update(nodes, aggs, u1a, u1b, Wun2, b, c)
    dec, hs = [], []
    for t in range(NSLAB):
        g2_t = _gather_pair(ps2, pd2, src[t], dst[t])
        dec_t, hp_t = _edge_step2(e1[t], g2_t, a, Wue2, Wd1, Wd2, blk)
        dec.append(dec_t)
        hs.append(jnp.sum(hp_t[:, 0, 0]))
    return jnp.concatenate(dec, axis=1).reshape(e, 1), sum(hs)


# speedup vs baseline: 5.0772x; 1.0076x over previous
import functools

import jax
import jax.numpy as jnp
from jax import lax
from jax.experimental import pallas as pl
from jax.experimental.pallas import tpu as pltpu
from jax.experimental.pallas import tpu_sc as plsc

LAT = 128
NCORES = 2
NSUB = 16
NW = NCORES * NSUB
CHUNK = 80

@functools.lru_cache(maxsize=1)
def _sc_mesh():
    return plsc.VectorSubcoreMesh(
        core_axis_name="c", subcore_axis_name="s",
        num_cores=NCORES, num_subcores=NSUB)


def _ln(h):
    mu = jnp.mean(h, axis=-1, keepdims=True)
    d = h - mu
    var = jnp.mean(d * d, axis=-1, keepdims=True)
    return d / jnp.sqrt(var + 1e-6)


def _relu(h):
    return jnp.maximum(h, 0.0)


def _node_encode_body(x_ref, wn1, wn2, b, c, nodes_ref, ps_ref, pd_ref):
    h = _relu(_relu(x_ref[...] @ wn1[...]) @ wn2[...])
    nodes = _ln(h)
    nodes_ref[...] = nodes
    ps_ref[...] = nodes @ b[...]
    pd_ref[...] = nodes @ c[...]


def _node_encode(x, wn1, wn2, b, c):
    n = x.shape[0]
    f = jax.ShapeDtypeStruct((n, LAT), jnp.float32)
    return pl.pallas_call(_node_encode_body, out_shape=(f, f, f))(x, wn1, wn2, b, c)


def _node_update_body(nodes_ref, *rest):
    aggs_refs = rest[:-7]
    u1a, u1b, wun2, b, c, ps_ref, pd_ref = rest[-7:]
    nodes = nodes_ref[...]
    n = nodes.shape[0]
    agg = sum(ar[i, :n, :] for ar in aggs_refs for i in range(NCORES))
    h = _relu(nodes @ u1a[...] + agg @ u1b[...])
    h = _relu(h @ wun2[...])
    n2 = nodes + _ln(h)
    ps_ref[...] = n2 @ b[...]
    pd_ref[...] = n2 @ c[...]


def _node_update(nodes, aggs, u1a, u1b, wun2, b, c):
    n = nodes.shape[0]
    f = jax.ShapeDtypeStruct((n, LAT), jnp.float32)
    return pl.pallas_call(_node_update_body, out_shape=(f, f))(
        nodes, *aggs, u1a, u1b, wun2, b, c)


def _bf(x):
    return x.astype(jnp.bfloat16)


def _mm(x, w):
    return lax.dot_general(_bf(x), _bf(w), (((1,), (0,)), ((), ())),
                           preferred_element_type=jnp.float32)


def _edge1_body(ea_ref, g_ref, we1, we2, a, wue2, e1_ref):
    h0 = lax.dot_general(_bf(ea_ref[...]), _bf(we1[...]),
                         (((0,), (0,)), ((), ())),
                         preferred_element_type=jnp.float32)
    e0 = _ln(_relu(_mm(_relu(h0), we2[...])))
    h = _relu(_mm(e0, a[...]) + g_ref[...])
    e1_ref[...] = e0 + _ln(_relu(_mm(h, wue2[...])))


def _edge_step1(ea_t, g, we1, we2, a, wue2, blk):
    de, e = ea_t.shape
    grid = (e // blk,)
    full = lambda i: (0, 0)
    return pl.pallas_call(
        _edge1_body,
        grid=grid,
        in_specs=[
            pl.BlockSpec((de, blk), lambda i: (0, i)),
            pl.BlockSpec((blk, LAT), lambda i: (i, 0)),
            pl.BlockSpec((de, LAT), full),
            pl.BlockSpec((LAT, LAT), full),
            pl.BlockSpec((LAT, LAT), full),
            pl.BlockSpec((LAT, LAT), full),
        ],
        out_specs=pl.BlockSpec((blk, LAT), lambda i: (i, 0)),
        out_shape=jax.ShapeDtypeStruct((e, LAT), jnp.float32),
        compiler_params=pltpu.CompilerParams(
            dimension_semantics=("arbitrary",)),
    )(ea_t, g, we1, we2, a, wue2)


def _edge2_body(e1_ref, g_ref, a, wue2, wd1, wd2, dec_ref, hp_ref):
    e1 = e1_ref[...]
    h = _relu(_mm(e1, a[...]) + g_ref[...])
    e2 = e1 + _ln(_relu(_mm(h, wue2[...])))
    h2 = _relu(_mm(e2, wd1[...]))
    d = lax.dot_general(_bf(wd2[...]), _bf(h2), (((0,), (1,)), ((), ())),
                        preferred_element_type=jnp.float32)
    dec_ref[...] = d
    hp_ref[...] = jnp.full((1, 1, LAT), jnp.sum(d), dtype=jnp.float32)


def _edge_step2(e1, g, a, wue2, wd1, wd2, blk):
    e = e1.shape[0]
    grid = (e // blk,)
    full = lambda i: (0, 0)
    return pl.pallas_call(
        _edge2_body,
        grid=grid,
        in_specs=[
            pl.BlockSpec((blk, LAT), lambda i: (i, 0)),
            pl.BlockSpec((blk, LAT), lambda i: (i, 0)),
            pl.BlockSpec((LAT, LAT), full),
            pl.BlockSpec((LAT, LAT), full),
            pl.BlockSpec((LAT, LAT), full),
            pl.BlockSpec((LAT, 1), full),
        ],
        out_specs=(
            pl.BlockSpec((1, blk), lambda i: (0, i)),
            pl.BlockSpec((1, 1, LAT), lambda i: (i, 0, 0)),
        ),
        out_shape=(
            jax.ShapeDtypeStruct((1, e), jnp.float32),
            jax.ShapeDtypeStruct((e // blk, 1, LAT), jnp.float32),
        ),
        compiler_params=pltpu.CompilerParams(
            dimension_semantics=("arbitrary",)),
    )(e1, g, a, wue2, wd1, wd2)


K2 = 128


def _gather_pair(ps, pd, src2, dst2):
    nw, ew = src2.shape
    e = nw * ew
    nch = ew // K2
    nchr = (nch // 2) * 2
    tail = ew - nch * K2

    @functools.partial(
        pl.kernel, mesh=_sc_mesh(),
        out_type=jax.ShapeDtypeStruct((e, LAT), jnp.float32),
        scratch_types=[
            pltpu.VMEM((ew,), jnp.int32),
            pltpu.VMEM((ew,), jnp.int32),
            pltpu.VMEM((2, K2, LAT), jnp.float32),
            pltpu.VMEM((2, K2, LAT), jnp.float32),
            pltpu.SemaphoreType.DMA,
            pltpu.SemaphoreType.DMA,
            pltpu.SemaphoreType.DMA,
            pltpu.SemaphoreType.DMA,
        ],
    )
    def kern(ps_hbm, pd_hbm, src_hbm, dst_hbm, out_hbm,
             idxs_v, idxd_v, bufp, bufd, g0, g1, w0, w1):
        wid = lax.axis_index("s") * NCORES + lax.axis_index("c")
        wbase = pl.multiple_of(wid * ew, 8)
        pltpu.sync_copy(src_hbm.at[wid], idxs_v)
        pltpu.sync_copy(dst_hbm.at[wid], idxd_v)
        gsem = (g0, g1)
        wsem = (w0, w1)

        def isl(v, i, k=K2):
            return v.at[pl.ds(pl.multiple_of(i * K2, 8), k)]

        def orows(i, k=K2):
            return out_hbm.at[pl.ds(pl.multiple_of(wbase + i * K2, 8), k)]

        def start_gather(i, b):
            pltpu.async_copy(ps_hbm.at[isl(idxs_v, i)], bufp.at[b], gsem[b])
            pltpu.async_copy(pd_hbm.at[isl(idxd_v, i)], bufd.at[b], gsem[b])

        def wait_gather(i, b):
            pltpu.make_async_copy(ps_hbm.at[isl(idxs_v, i)], bufp.at[b], gsem[b]).wait()
            pltpu.make_async_copy(pd_hbm.at[isl(idxd_v, i)], bufd.at[b], gsem[b]).wait()

        def add_buf(b, k):
            def addrow(r, carry):
                for j in range(LAT // 16):
                    sl = pl.ds(j * 16, 16)
                    bufp[b, r, sl] = bufp[b, r, sl] + bufd[b, r, sl]
                return carry
            lax.fori_loop(0, k, addrow, 0)

        start_gather(0, 0)
        start_gather(1, 1)

        def body(g, carry):
            for b in range(2):
                i = 2 * g + b
                wait_gather(i, b)
                add_buf(b, K2)
                pltpu.async_copy(bufp.at[b], orows(i), wsem[b])
            for b in range(2):
                i = 2 * g + b

                @pl.when(i + 2 < nchr)
                def _():
                    pltpu.make_async_copy(bufp.at[b], orows(i), wsem[b]).wait()
                    start_gather(i + 2, b)
            return carry
        lax.fori_loop(0, nchr // 2, body, 0)
        for b in range(2):
            pltpu.make_async_copy(bufp.at[b], orows(nchr - 2 + b), wsem[b]).wait()
        if nch > nchr:
            start_gather(nchr, 0)
            wait_gather(nchr, 0)
            add_buf(0, K2)
            pltpu.sync_copy(bufp.at[0], orows(nchr))
        if tail:
            cps = pltpu.async_copy(
                ps_hbm.at[isl(idxs_v, nch, tail)], bufp.at[0].at[pl.ds(0, tail)], g0)
            cpd = pltpu.async_copy(
                pd_hbm.at[isl(idxd_v, nch, tail)], bufd.at[0].at[pl.ds(0, tail)], g0)
            cps.wait()
            cpd.wait()
            add_buf(0, tail)
            pltpu.sync_copy(bufp.at[0].at[pl.ds(0, tail)], orows(nch, tail))

    return kern(ps, pd, src2, dst2)


def _segment_sum(e1, dstp, ew):
    nw, nchp, ks = dstp.shape
    nch = nchp - 1
    nchr = (nch // 2) * 2
    tail = ew - nch * ks
    npad = NPAD
    rpt = npad // NSUB
    stg = 64

    @functools.partial(
        pl.kernel, mesh=_sc_mesh(),
        out_type=jax.ShapeDtypeStruct((NCORES, npad, LAT), jnp.float32),
        scratch_types=[
            pltpu.VMEM((nchp, ks), jnp.int32),
            pltpu.VMEM((2, ks, LAT), jnp.float32),
            pltpu.VMEM((stg, LAT), jnp.float32),
            pltpu.VMEM_SHARED((npad, LAT), jnp.float32),
            pltpu.SemaphoreType.DMA,
            pltpu.SemaphoreType.DMA,
        ],
    )
    def kern(e_hbm, dst_hbm, out_hbm, idxd_v, rows, stage, acc_sh, r0, r1):
        c = lax.axis_index("c")
        s = lax.axis_index("s")
        wid = s * NCORES + c
        wbase = pl.multiple_of(wid * ew, 8)
        rsem = (r0, r1)

        def erows(i, k=ks):
            return e_hbm.at[pl.ds(pl.multiple_of(wbase + i * ks, 8), k)]

        pltpu.sync_copy(dst_hbm.at[wid], idxd_v)
        pltpu.async_copy(erows(0), rows.at[0], r0)
        pltpu.async_copy(erows(1), rows.at[1], r1)

        def zrows(ref, lo, cnt):
            def zrow(r, carry):
                for j in range(LAT // 16):
                    ref[r, pl.ds(j * 16, 16)] = jnp.zeros((16,), jnp.float32)
                return carry
            lax.fori_loop(lo, lo + cnt, zrow, 0)

        zrows(stage, 0, stg)
        for t in range(rpt // stg):
            pltpu.sync_copy(
                stage, acc_sh.at[pl.ds(pl.multiple_of(s * rpt + t * stg, 8), stg)])
        plsc.subcore_barrier()

        def body(g, carry):
            for b in range(2):
                i = 2 * g + b
                pltpu.make_async_copy(erows(i), rows.at[b], rsem[b]).wait()
                pltpu.sync_copy(rows.at[b], acc_sh.at[idxd_v.at[i]], add=True)

                @pl.when(i + 2 < nchr)
                def _():
                    pltpu.async_copy(erows(i + 2), rows.at[b], rsem[b])
            return carry
        lax.fori_loop(0, nchr // 2, body, 0)
        if nch > nchr:
            pltpu.async_copy(erows(nchr), rows.at[0], r0).wait()
            pltpu.sync_copy(rows.at[0], acc_sh.at[idxd_v.at[nchr]], add=True)
        pltpu.async_copy(erows(nch, tail), rows.at[0].at[pl.ds(0, tail)], r0).wait()
        def zrow2(r, carry):
            for j in range(LAT // 16):
                rows[0, r, pl.ds(j * 16, 16)] = jnp.zeros((16,), jnp.float32)
            return carry
        lax.fori_loop(tail, ks, zrow2, 0)
        pltpu.sync_copy(rows.at[0], acc_sh.at[idxd_v.at[nch]], add=True)
        plsc.subcore_barrier()

        for t in range(rpt // stg):
            sl = pl.ds(pl.multiple_of(s * rpt + t * stg, 8), stg)
            pltpu.sync_copy(acc_sh.at[sl], stage)
            pltpu.sync_copy(stage, out_hbm.at[c].at[sl])

    return kern(e1, dstp)


NSLAB = 2
NPAD = 10240


def kernel(x, edge_index, edge_attr, Wn1, Wn2, We1, We2,
           Wue1, Wue2, Wun1, Wun2, Wd1, Wd2):
    n = x.shape[0]
    e = edge_attr.shape[0]
    blk = 3200
    es = e // NSLAB
    ews = es // NW

    src = edge_index[0].astype(jnp.int32).reshape(NSLAB, NW, ews)
    dst = edge_index[1].astype(jnp.int32).reshape(NSLAB, NW, ews)
    ks = 64
    nchs = ews // ks
    pad = (nchs + 1) * ks - ews
    dstp = jnp.concatenate(
        [dst, jnp.full((NSLAB, NW, pad), NPAD - 1, jnp.int32)], axis=2
    ).reshape(NSLAB, NW, nchs + 1, ks)
    a, b, c = Wue1[:LAT], Wue1[LAT:2 * LAT], Wue1[2 * LAT:]
    u1a, u1b = Wun1[:LAT], Wun1[LAT:]

    nodes, ps, pd = _node_encode(x, Wn1, Wn2, b, c)
    g1 = [_gather_pair(ps, pd, src[t], dst[t]) for t in range(NSLAB)]
    e1, aggs = [], []
    for t in range(NSLAB):
        ea_t = lax.slice_in_dim(edge_attr.T, t * es, (t + 1) * es, axis=1)
        e1.append(_edge_step1(ea_t, g1[t], We1, We2, a, Wue2, blk))
        aggs.append(_segment_sum(e1[t], dstp[t], ews))
    ps2, pd2 = _node_update(nodes, aggs, u1a, u1b, Wun2, b, c)
    dec, hs = [], []
    for t in range(NSLAB):
        g2_t = _gather_pair(ps2, pd2, src[t], dst[t])
        dec_t, hp_t = _edge_step2(e1[t], g2_t, a, Wue2, Wd1, Wd2, blk)
        dec.append(dec_t)
        hs.append(jnp.sum(hp_t[:, 0, 0]))
    return jnp.concatenate(dec, axis=1).reshape(e, 1), sum(hs)


# speedup vs baseline: 5.1187x; 1.0082x over previous
import functools

import jax
import jax.numpy as jnp
from jax import lax
from jax.experimental import pallas as pl
from jax.experimental.pallas import tpu as pltpu
from jax.experimental.pallas import tpu_sc as plsc

LAT = 128
NCORES = 2
NSUB = 16
NW = NCORES * NSUB
CHUNK = 80

@functools.lru_cache(maxsize=1)
def _sc_mesh():
    return plsc.VectorSubcoreMesh(
        core_axis_name="c", subcore_axis_name="s",
        num_cores=NCORES, num_subcores=NSUB)


def _ln(h):
    mu = jnp.mean(h, axis=-1, keepdims=True)
    d = h - mu
    var = jnp.mean(d * d, axis=-1, keepdims=True)
    return d / jnp.sqrt(var + 1e-6)


def _relu(h):
    return jnp.maximum(h, 0.0)


def _node_encode_body(x_ref, wn1, wn2, b, c, nodes_ref, ps_ref, pd_ref):
    h = _relu(_relu(x_ref[...] @ wn1[...]) @ wn2[...])
    nodes = _ln(h)
    nodes_ref[...] = nodes
    ps_ref[...] = nodes @ b[...]
    pd_ref[...] = nodes @ c[...]


def _node_encode(x, wn1, wn2, b, c):
    n = x.shape[0]
    f = jax.ShapeDtypeStruct((n, LAT), jnp.float32)
    return pl.pallas_call(_node_encode_body, out_shape=(f, f, f))(x, wn1, wn2, b, c)


def _node_update_body(nodes_ref, *rest):
    aggs_refs = rest[:-7]
    u1a, u1b, wun2, b, c, ps_ref, pd_ref = rest[-7:]
    nodes = nodes_ref[...]
    n = nodes.shape[0]
    agg = sum(ar[i, :n, :] for ar in aggs_refs for i in range(NCORES))
    h = _relu(nodes @ u1a[...] + agg @ u1b[...])
    h = _relu(h @ wun2[...])
    n2 = nodes + _ln(h)
    ps_ref[...] = n2 @ b[...]
    pd_ref[...] = n2 @ c[...]


def _node_update(nodes, aggs, u1a, u1b, wun2, b, c):
    n = nodes.shape[0]
    f = jax.ShapeDtypeStruct((n, LAT), jnp.float32)
    return pl.pallas_call(_node_update_body, out_shape=(f, f))(
        nodes, *aggs, u1a, u1b, wun2, b, c)


def _bf(x):
    return x.astype(jnp.bfloat16)


def _mm(x, w):
    return lax.dot_general(_bf(x), _bf(w), (((1,), (0,)), ((), ())),
                           preferred_element_type=jnp.float32)


def _edge1_body(ea_ref, g_ref, we1, we2, a, wue2, e1_ref):
    h0 = lax.dot_general(_bf(ea_ref[...]), _bf(we1[...]),
                         (((0,), (0,)), ((), ())),
                         preferred_element_type=jnp.float32)
    e0 = _ln(_relu(_mm(_relu(h0), we2[...])))
    h = _relu(_mm(e0, a[...]) + g_ref[...])
    e1_ref[...] = e0 + _ln(_relu(_mm(h, wue2[...])))


def _edge_step1(ea_t, g, we1, we2, a, wue2, blk):
    de, e = ea_t.shape
    grid = (e // blk,)
    full = lambda i: (0, 0)
    return pl.pallas_call(
        _edge1_body,
        grid=grid,
        in_specs=[
            pl.BlockSpec((de, blk), lambda i: (0, i)),
            pl.BlockSpec((blk, LAT), lambda i: (i, 0)),
            pl.BlockSpec((de, LAT), full),
            pl.BlockSpec((LAT, LAT), full),
            pl.BlockSpec((LAT, LAT), full),
            pl.BlockSpec((LAT, LAT), full),
        ],
        out_specs=pl.BlockSpec((blk, LAT), lambda i: (i, 0)),
        out_shape=jax.ShapeDtypeStruct((e, LAT), jnp.float32),
        compiler_params=pltpu.CompilerParams(
            dimension_semantics=("arbitrary",)),
    )(ea_t, g, we1, we2, a, wue2)


def _edge2_body(e1_ref, g_ref, a, wue2, wd1, wd2, dec_ref, hp_ref):
    e1 = e1_ref[...]
    h = _relu(_mm(e1, a[...]) + g_ref[...])
    e2 = e1 + _ln(_relu(_mm(h, wue2[...])))
    h2 = _relu(_mm(e2, wd1[...]))
    d = lax.dot_general(_bf(wd2[...]), _bf(h2), (((0,), (1,)), ((), ())),
                        preferred_element_type=jnp.float32)
    dec_ref[...] = d
    hp_ref[...] = jnp.full((1, 1, LAT), jnp.sum(d), dtype=jnp.float32)


def _edge_step2(e1, g, a, wue2, wd1, wd2, blk):
    e = e1.shape[0]
    grid = (e // blk,)
    full = lambda i: (0, 0)
    return pl.pallas_call(
        _edge2_body,
        grid=grid,
        in_specs=[
            pl.BlockSpec((blk, LAT), lambda i: (i, 0)),
            pl.BlockSpec((blk, LAT), lambda i: (i, 0)),
            pl.BlockSpec((LAT, LAT), full),
            pl.BlockSpec((LAT, LAT), full),
            pl.BlockSpec((LAT, LAT), full),
            pl.BlockSpec((LAT, 1), full),
        ],
        out_specs=(
            pl.BlockSpec((1, blk), lambda i: (0, i)),
            pl.BlockSpec((1, 1, LAT), lambda i: (i, 0, 0)),
        ),
        out_shape=(
            jax.ShapeDtypeStruct((1, e), jnp.float32),
            jax.ShapeDtypeStruct((e // blk, 1, LAT), jnp.float32),
        ),
        compiler_params=pltpu.CompilerParams(
            dimension_semantics=("arbitrary",)),
    )(e1, g, a, wue2, wd1, wd2)


K2 = 128


def _gather_pair(ps, pd, src2, dst2):
    nw, ew = src2.shape
    e = nw * ew
    nch = ew // K2
    nchr = (nch // 2) * 2
    tail = ew - nch * K2

    @functools.partial(
        pl.kernel, mesh=_sc_mesh(),
        out_type=jax.ShapeDtypeStruct((e, LAT), jnp.float32),
        scratch_types=[
            pltpu.VMEM((ew,), jnp.int32),
            pltpu.VMEM((ew,), jnp.int32),
            pltpu.VMEM((2, K2, LAT), jnp.float32),
            pltpu.VMEM((2, K2, LAT), jnp.float32),
            pltpu.SemaphoreType.DMA,
            pltpu.SemaphoreType.DMA,
            pltpu.SemaphoreType.DMA,
            pltpu.SemaphoreType.DMA,
        ],
    )
    def kern(ps_hbm, pd_hbm, src_hbm, dst_hbm, out_hbm,
             idxs_v, idxd_v, bufp, bufd, g0, g1, w0, w1):
        wid = lax.axis_index("s") * NCORES + lax.axis_index("c")
        wbase = pl.multiple_of(wid * ew, 8)
        pltpu.sync_copy(src_hbm.at[wid], idxs_v)
        pltpu.sync_copy(dst_hbm.at[wid], idxd_v)
        gsem = (g0, g1)
        wsem = (w0, w1)

        def isl(v, i, k=K2):
            return v.at[pl.ds(pl.multiple_of(i * K2, 8), k)]

        def orows(i, k=K2):
            return out_hbm.at[pl.ds(pl.multiple_of(wbase + i * K2, 8), k)]

        def start_gather(i, b):
            pltpu.async_copy(ps_hbm.at[isl(idxs_v, i)], bufp.at[b], gsem[b])
            pltpu.async_copy(pd_hbm.at[isl(idxd_v, i)], bufd.at[b], gsem[b])

        def wait_gather(i, b):
            pltpu.make_async_copy(ps_hbm.at[isl(idxs_v, i)], bufp.at[b], gsem[b]).wait()
            pltpu.make_async_copy(pd_hbm.at[isl(idxd_v, i)], bufd.at[b], gsem[b]).wait()

        def add_buf(b, k):
            def addrow(r, carry):
                for j in range(LAT // 16):
                    sl = pl.ds(j * 16, 16)
                    bufp[b, r, sl] = bufp[b, r, sl] + bufd[b, r, sl]
                return carry
            lax.fori_loop(0, k, addrow, 0)

        start_gather(0, 0)
        start_gather(1, 1)

        def body(g, carry):
            for b in range(2):
                i = 2 * g + b
                wait_gather(i, b)
                add_buf(b, K2)
                pltpu.async_copy(bufp.at[b], orows(i), wsem[b])
            for b in range(2):
                i = 2 * g + b

                @pl.when(i + 2 < nchr)
                def _():
                    pltpu.make_async_copy(bufp.at[b], orows(i), wsem[b]).wait()
                    start_gather(i + 2, b)
            return carry
        lax.fori_loop(0, nchr // 2, body, 0)
        for b in range(2):
            pltpu.make_async_copy(bufp.at[b], orows(nchr - 2 + b), wsem[b]).wait()
        if nch > nchr:
            start_gather(nchr, 0)
            wait_gather(nchr, 0)
            add_buf(0, K2)
            pltpu.sync_copy(bufp.at[0], orows(nchr))
        if tail:
            cps = pltpu.async_copy(
                ps_hbm.at[isl(idxs_v, nch, tail)], bufp.at[0].at[pl.ds(0, tail)], g0)
            cpd = pltpu.async_copy(
                pd_hbm.at[isl(idxd_v, nch, tail)], bufd.at[0].at[pl.ds(0, tail)], g0)
            cps.wait()
            cpd.wait()
            add_buf(0, tail)
            pltpu.sync_copy(bufp.at[0].at[pl.ds(0, tail)], orows(nch, tail))

    return kern(ps, pd, src2, dst2)


def _segment_sum(e1, dstp, ew):
    nw, nchp, ks = dstp.shape
    nch = nchp - 1
    nchr = (nch // 2) * 2
    tail = ew - nch * ks
    npad = NPAD
    rpt = npad // NSUB
    stg = 64

    @functools.partial(
        pl.kernel, mesh=_sc_mesh(),
        out_type=jax.ShapeDtypeStruct((NCORES, npad, LAT), jnp.float32),
        scratch_types=[
            pltpu.VMEM((nchp, ks), jnp.int32),
            pltpu.VMEM((2, ks, LAT), jnp.float32),
            pltpu.VMEM((stg, LAT), jnp.float32),
            pltpu.VMEM_SHARED((npad, LAT), jnp.float32),
            pltpu.SemaphoreType.DMA,
            pltpu.SemaphoreType.DMA,
        ],
    )
    def kern(e_hbm, dst_hbm, out_hbm, idxd_v, rows, stage, acc_sh, r0, r1):
        c = lax.axis_index("c")
        s = lax.axis_index("s")
        wid = s * NCORES + c
        wbase = pl.multiple_of(wid * ew, 8)
        rsem = (r0, r1)

        def erows(i, k=ks):
            return e_hbm.at[pl.ds(pl.multiple_of(wbase + i * ks, 8), k)]

        pltpu.sync_copy(dst_hbm.at[wid], idxd_v)
        pltpu.async_copy(erows(0), rows.at[0], r0)
        pltpu.async_copy(erows(1), rows.at[1], r1)

        def zrows(ref, lo, cnt):
            def zrow(r, carry):
                for j in range(LAT // 16):
                    ref[r, pl.ds(j * 16, 16)] = jnp.zeros((16,), jnp.float32)
                return carry
            lax.fori_loop(lo, lo + cnt, zrow, 0)

        zrows(stage, 0, stg)
        for t in range(rpt // stg):
            pltpu.sync_copy(
                stage, acc_sh.at[pl.ds(pl.multiple_of(s * rpt + t * stg, 8), stg)])
        plsc.subcore_barrier()

        def body(g, carry):
            for b in range(2):
                i = 2 * g + b
                pltpu.make_async_copy(erows(i), rows.at[b], rsem[b]).wait()
                pltpu.sync_copy(rows.at[b], acc_sh.at[idxd_v.at[i]], add=True)

                @pl.when(i + 2 < nchr)
                def _():
                    pltpu.async_copy(erows(i + 2), rows.at[b], rsem[b])
            return carry
        lax.fori_loop(0, nchr // 2, body, 0)
        if nch > nchr:
            pltpu.async_copy(erows(nchr), rows.at[0], r0).wait()
            pltpu.sync_copy(rows.at[0], acc_sh.at[idxd_v.at[nchr]], add=True)
        pltpu.async_copy(erows(nch, tail), rows.at[0].at[pl.ds(0, tail)], r0).wait()
        def zrow2(r, carry):
            for j in range(LAT // 16):
                rows[0, r, pl.ds(j * 16, 16)] = jnp.zeros((16,), jnp.float32)
            return carry
        lax.fori_loop(tail, ks, zrow2, 0)
        pltpu.sync_copy(rows.at[0], acc_sh.at[idxd_v.at[nch]], add=True)
        plsc.subcore_barrier()

        for t in range(rpt // stg):
            sl = pl.ds(pl.multiple_of(s * rpt + t * stg, 8), stg)
            pltpu.sync_copy(acc_sh.at[sl], stage)
            pltpu.sync_copy(stage, out_hbm.at[c].at[sl])

    return kern(e1, dstp)


SLAB_SIZES = (76800, 102400, 140800)
NPAD = 10240


def kernel(x, edge_index, edge_attr, Wn1, Wn2, We1, We2,
           Wue1, Wue2, Wun1, Wun2, Wd1, Wd2):
    n = x.shape[0]
    e = edge_attr.shape[0]
    blk = 3200
    ks = 64

    src_fl = edge_index[0].astype(jnp.int32)
    dst_fl = edge_index[1].astype(jnp.int32)
    ea_t_full = edge_attr.T
    offs = [0]
    for sz in SLAB_SIZES[:-1]:
        offs.append(offs[-1] + sz)

    src2, dst2, dstp, ea_t = [], [], [], []
    for o, sz in zip(offs, SLAB_SIZES):
        ews = sz // NW
        src2.append(lax.slice_in_dim(src_fl, o, o + sz).reshape(NW, ews))
        d2 = lax.slice_in_dim(dst_fl, o, o + sz).reshape(NW, ews)
        dst2.append(d2)
        nchs = ews // ks
        pad = (nchs + 1) * ks - ews
        dstp.append(jnp.concatenate(
            [d2, jnp.full((NW, pad), NPAD - 1, jnp.int32)], axis=1
        ).reshape(NW, nchs + 1, ks))
        ea_t.append(lax.slice_in_dim(ea_t_full, o, o + sz, axis=1))

    a, b, c = Wue1[:LAT], Wue1[LAT:2 * LAT], Wue1[2 * LAT:]
    u1a, u1b = Wun1[:LAT], Wun1[LAT:]
    ns = len(SLAB_SIZES)

    nodes, ps, pd = _node_encode(x, Wn1, Wn2, b, c)
    g1 = [_gather_pair(ps, pd, src2[t], dst2[t]) for t in range(ns)]
    e1, aggs = [], []
    for t in range(ns):
        e1.append(_edge_step1(ea_t[t], g1[t], We1, We2, a, Wue2, blk))
        aggs.append(_segment_sum(e1[t], dstp[t], SLAB_SIZES[t] // NW))
    ps2, pd2 = _node_update(nodes, aggs, u1a, u1b, Wun2, b, c)
    dec, hs = [], []
    for t in range(ns):
        g2_t = _gather_pair(ps2, pd2, src2[t], dst2[t])
        dec_t, hp_t = _edge_step2(e1[t], g2_t, a, Wue2, Wd1, Wd2, blk)
        dec.append(dec_t)
        hs.append(jnp.sum(hp_t[:, 0, 0]))
    return jnp.concatenate(dec, axis=1).reshape(e, 1), sum(hs)


# speedup vs baseline: 5.2271x; 1.0212x over previous
import functools

import jax
import jax.numpy as jnp
from jax import lax
from jax.experimental import pallas as pl
from jax.experimental.pallas import tpu as pltpu
from jax.experimental.pallas import tpu_sc as plsc

LAT = 128
NCORES = 2
NSUB = 16
NW = NCORES * NSUB
CHUNK = 80

@functools.lru_cache(maxsize=1)
def _sc_mesh():
    return plsc.VectorSubcoreMesh(
        core_axis_name="c", subcore_axis_name="s",
        num_cores=NCORES, num_subcores=NSUB)


def _ln(h):
    mu = jnp.mean(h, axis=-1, keepdims=True)
    d = h - mu
    var = jnp.mean(d * d, axis=-1, keepdims=True)
    return d / jnp.sqrt(var + 1e-6)


def _relu(h):
    return jnp.maximum(h, 0.0)


def _node_encode_body(x_ref, wn1, wn2, b, c, nodes_ref, ps_ref, pd_ref):
    h = _relu(_relu(x_ref[...] @ wn1[...]) @ wn2[...])
    nodes = _ln(h)
    nodes_ref[...] = nodes
    ps_ref[...] = nodes @ b[...]
    pd_ref[...] = nodes @ c[...]


def _node_encode(x, wn1, wn2, b, c):
    n = x.shape[0]
    f = jax.ShapeDtypeStruct((n, LAT), jnp.float32)
    return pl.pallas_call(_node_encode_body, out_shape=(f, f, f))(x, wn1, wn2, b, c)


def _node_update_body(nodes_ref, *rest):
    aggs_refs = rest[:-7]
    u1a, u1b, wun2, b, c, ps_ref, pd_ref = rest[-7:]
    nodes = nodes_ref[...]
    n = nodes.shape[0]
    agg = sum(ar[i, :n, :] for ar in aggs_refs for i in range(NCORES))
    h = _relu(nodes @ u1a[...] + agg @ u1b[...])
    h = _relu(h @ wun2[...])
    n2 = nodes + _ln(h)
    ps_ref[...] = n2 @ b[...]
    pd_ref[...] = n2 @ c[...]


def _node_update(nodes, aggs, u1a, u1b, wun2, b, c):
    n = nodes.shape[0]
    bn = 2000
    f = jax.ShapeDtypeStruct((n, LAT), jnp.float32)
    full = lambda i: (0, 0)
    return pl.pallas_call(
        _node_update_body,
        grid=(n // bn,),
        in_specs=[pl.BlockSpec((bn, LAT), lambda i: (i, 0))]
        + [pl.BlockSpec((NCORES, bn, LAT), lambda i: (0, i, 0))] * len(aggs)
        + [pl.BlockSpec((LAT, LAT), full)] * 5,
        out_specs=(pl.BlockSpec((bn, LAT), lambda i: (i, 0)),
                   pl.BlockSpec((bn, LAT), lambda i: (i, 0))),
        out_shape=(f, f),
    )(nodes, *aggs, u1a, u1b, wun2, b, c)


def _bf(x):
    return x.astype(jnp.bfloat16)


def _mm(x, w):
    return lax.dot_general(_bf(x), _bf(w), (((1,), (0,)), ((), ())),
                           preferred_element_type=jnp.float32)


def _edge1_body(ea_ref, g_ref, we1, we2, a, wue2, e1_ref):
    h0 = lax.dot_general(_bf(ea_ref[...]), _bf(we1[...]),
                         (((0,), (0,)), ((), ())),
                         preferred_element_type=jnp.float32)
    e0 = _ln(_relu(_mm(_relu(h0), we2[...])))
    h = _relu(_mm(e0, a[...]) + g_ref[...])
    e1_ref[...] = e0 + _ln(_relu(_mm(h, wue2[...])))


def _edge_step1(ea_t_full, off, g, we1, we2, a, wue2, blk):
    de = ea_t_full.shape[0]
    e = g.shape[0]
    grid = (e // blk,)
    ob = off // blk
    full = lambda i: (0, 0)
    return pl.pallas_call(
        _edge1_body,
        grid=grid,
        in_specs=[
            pl.BlockSpec((de, blk), lambda i: (0, i + ob)),
            pl.BlockSpec((blk, LAT), lambda i: (i, 0)),
            pl.BlockSpec((de, LAT), full),
            pl.BlockSpec((LAT, LAT), full),
            pl.BlockSpec((LAT, LAT), full),
            pl.BlockSpec((LAT, LAT), full),
        ],
        out_specs=pl.BlockSpec((blk, LAT), lambda i: (i, 0)),
        out_shape=jax.ShapeDtypeStruct((e, LAT), jnp.float32),
        compiler_params=pltpu.CompilerParams(
            dimension_semantics=("arbitrary",)),
    )(ea_t_full, g, we1, we2, a, wue2)


def _edge2_body(e1_ref, g_ref, a, wue2, wd1, wd2, dec_ref, hp_ref):
    e1 = e1_ref[...]
    h = _relu(_mm(e1, a[...]) + g_ref[...])
    e2 = e1 + _ln(_relu(_mm(h, wue2[...])))
    h2 = _relu(_mm(e2, wd1[...]))
    d = lax.dot_general(_bf(wd2[...]), _bf(h2), (((0,), (1,)), ((), ())),
                        preferred_element_type=jnp.float32)
    dec_ref[...] = d
    hp_ref[...] = jnp.full((1, 1, LAT), jnp.sum(d), dtype=jnp.float32)


def _edge_step2(e1, g, a, wue2, wd1, wd2, blk):
    e = e1.shape[0]
    grid = (e // blk,)
    full = lambda i: (0, 0)
    return pl.pallas_call(
        _edge2_body,
        grid=grid,
        in_specs=[
            pl.BlockSpec((blk, LAT), lambda i: (i, 0)),
            pl.BlockSpec((blk, LAT), lambda i: (i, 0)),
            pl.BlockSpec((LAT, LAT), full),
            pl.BlockSpec((LAT, LAT), full),
            pl.BlockSpec((LAT, LAT), full),
            pl.BlockSpec((LAT, 1), full),
        ],
        out_specs=(
            pl.BlockSpec((1, blk), lambda i: (0, i)),
            pl.BlockSpec((1, 1, LAT), lambda i: (i, 0, 0)),
        ),
        out_shape=(
            jax.ShapeDtypeStruct((1, e), jnp.float32),
            jax.ShapeDtypeStruct((e // blk, 1, LAT), jnp.float32),
        ),
        compiler_params=pltpu.CompilerParams(
            dimension_semantics=("arbitrary",)),
    )(e1, g, a, wue2, wd1, wd2)


K2 = 128


def _gather_pair(ps, pd, src2, dst2):
    nw, ew = src2.shape
    e = nw * ew
    nch = ew // K2
    nchr = (nch // 2) * 2
    tail = ew - nch * K2

    @functools.partial(
        pl.kernel, mesh=_sc_mesh(),
        out_type=jax.ShapeDtypeStruct((e, LAT), jnp.float32),
        scratch_types=[
            pltpu.VMEM((ew,), jnp.int32),
            pltpu.VMEM((ew,), jnp.int32),
            pltpu.VMEM((2, K2, LAT), jnp.float32),
            pltpu.VMEM((2, K2, LAT), jnp.float32),
            pltpu.SemaphoreType.DMA,
            pltpu.SemaphoreType.DMA,
            pltpu.SemaphoreType.DMA,
            pltpu.SemaphoreType.DMA,
        ],
    )
    def kern(ps_hbm, pd_hbm, src_hbm, dst_hbm, out_hbm,
             idxs_v, idxd_v, bufp, bufd, g0, g1, w0, w1):
        wid = lax.axis_index("s") * NCORES + lax.axis_index("c")
        wbase = pl.multiple_of(wid * ew, 8)
        pltpu.sync_copy(src_hbm.at[wid], idxs_v)
        pltpu.sync_copy(dst_hbm.at[wid], idxd_v)
        gsem = (g0, g1)
        wsem = (w0, w1)

        def isl(v, i, k=K2):
            return v.at[pl.ds(pl.multiple_of(i * K2, 8), k)]

        def orows(i, k=K2):
            return out_hbm.at[pl.ds(pl.multiple_of(wbase + i * K2, 8), k)]

        def start_gather(i, b):
            pltpu.async_copy(ps_hbm.at[isl(idxs_v, i)], bufp.at[b], gsem[b])
            pltpu.async_copy(pd_hbm.at[isl(idxd_v, i)], bufd.at[b], gsem[b])

        def wait_gather(i, b):
            pltpu.make_async_copy(ps_hbm.at[isl(idxs_v, i)], bufp.at[b], gsem[b]).wait()
            pltpu.make_async_copy(pd_hbm.at[isl(idxd_v, i)], bufd.at[b], gsem[b]).wait()

        def add_buf(b, k):
            def addrow(r, carry):
                for j in range(LAT // 16):
                    sl = pl.ds(j * 16, 16)
                    bufp[b, r, sl] = bufp[b, r, sl] + bufd[b, r, sl]
                return carry
            lax.fori_loop(0, k, addrow, 0)

        start_gather(0, 0)
        start_gather(1, 1)

        def body(g, carry):
            for b in range(2):
                i = 2 * g + b
                wait_gather(i, b)
                add_buf(b, K2)
                pltpu.async_copy(bufp.at[b], orows(i), wsem[b])
            for b in range(2):
                i = 2 * g + b

                @pl.when(i + 2 < nchr)
                def _():
                    pltpu.make_async_copy(bufp.at[b], orows(i), wsem[b]).wait()
                    start_gather(i + 2, b)
            return carry
        lax.fori_loop(0, nchr // 2, body, 0)
        for b in range(2):
            pltpu.make_async_copy(bufp.at[b], orows(nchr - 2 + b), wsem[b]).wait()
        if nch > nchr:
            start_gather(nchr, 0)
            wait_gather(nchr, 0)
            add_buf(0, K2)
            pltpu.sync_copy(bufp.at[0], orows(nchr))
        if tail:
            cps = pltpu.async_copy(
                ps_hbm.at[isl(idxs_v, nch, tail)], bufp.at[0].at[pl.ds(0, tail)], g0)
            cpd = pltpu.async_copy(
                pd_hbm.at[isl(idxd_v, nch, tail)], bufd.at[0].at[pl.ds(0, tail)], g0)
            cps.wait()
            cpd.wait()
            add_buf(0, tail)
            pltpu.sync_copy(bufp.at[0].at[pl.ds(0, tail)], orows(nch, tail))

    return kern(ps, pd, src2, dst2)


def _segment_sum(e1, dstp, ew):
    nw, nchp, ks = dstp.shape
    nch = nchp - 1
    nchr = (nch // 2) * 2
    tail = ew - nch * ks
    npad = NPAD
    rpt = npad // NSUB
    stg = 64

    @functools.partial(
        pl.kernel, mesh=_sc_mesh(),
        out_type=jax.ShapeDtypeStruct((NCORES, npad, LAT), jnp.float32),
        scratch_types=[
            pltpu.VMEM((nchp, ks), jnp.int32),
            pltpu.VMEM((2, ks, LAT), jnp.float32),
            pltpu.VMEM((stg, LAT), jnp.float32),
            pltpu.VMEM_SHARED((npad, LAT), jnp.float32),
            pltpu.SemaphoreType.DMA,
            pltpu.SemaphoreType.DMA,
        ],
    )
    def kern(e_hbm, dst_hbm, out_hbm, idxd_v, rows, stage, acc_sh, r0, r1):
        c = lax.axis_index("c")
        s = lax.axis_index("s")
        wid = s * NCORES + c
        wbase = pl.multiple_of(wid * ew, 8)
        rsem = (r0, r1)

        def erows(i, k=ks):
            return e_hbm.at[pl.ds(pl.multiple_of(wbase + i * ks, 8), k)]

        pltpu.sync_copy(dst_hbm.at[wid], idxd_v)
        pltpu.async_copy(erows(0), rows.at[0], r0)
        pltpu.async_copy(erows(1), rows.at[1], r1)

        def zrows(ref, lo, cnt):
            def zrow(r, carry):
                for j in range(LAT // 16):
                    ref[r, pl.ds(j * 16, 16)] = jnp.zeros((16,), jnp.float32)
                return carry
            lax.fori_loop(lo, lo + cnt, zrow, 0)

        zrows(stage, 0, stg)
        for t in range(rpt // stg):
            pltpu.sync_copy(
                stage, acc_sh.at[pl.ds(pl.multiple_of(s * rpt + t * stg, 8), stg)])
        plsc.subcore_barrier()

        def body(g, carry):
            for b in range(2):
                i = 2 * g + b
                pltpu.make_async_copy(erows(i), rows.at[b], rsem[b]).wait()
                pltpu.sync_copy(rows.at[b], acc_sh.at[idxd_v.at[i]], add=True)

                @pl.when(i + 2 < nchr)
                def _():
                    pltpu.async_copy(erows(i + 2), rows.at[b], rsem[b])
            return carry
        lax.fori_loop(0, nchr // 2, body, 0)
        if nch > nchr:
            pltpu.async_copy(erows(nchr), rows.at[0], r0).wait()
            pltpu.sync_copy(rows.at[0], acc_sh.at[idxd_v.at[nchr]], add=True)
        pltpu.async_copy(erows(nch, tail), rows.at[0].at[pl.ds(0, tail)], r0).wait()
        def zrow2(r, carry):
            for j in range(LAT // 16):
                rows[0, r, pl.ds(j * 16, 16)] = jnp.zeros((16,), jnp.float32)
            return carry
        lax.fori_loop(tail, ks, zrow2, 0)
        pltpu.sync_copy(rows.at[0], acc_sh.at[idxd_v.at[nch]], add=True)
        plsc.subcore_barrier()

        for t in range(rpt // stg):
            sl = pl.ds(pl.multiple_of(s * rpt + t * stg, 8), stg)
            pltpu.sync_copy(acc_sh.at[sl], stage)
            pltpu.sync_copy(stage, out_hbm.at[c].at[sl])

    return kern(e1, dstp)


SLAB_SIZES = (51200, 76800, 89600, 102400)
NPAD = 10240


def kernel(x, edge_index, edge_attr, Wn1, Wn2, We1, We2,
           Wue1, Wue2, Wun1, Wun2, Wd1, Wd2):
    n = x.shape[0]
    e = edge_attr.shape[0]
    blk = 3200
    ks = 64

    src_fl = edge_index[0].astype(jnp.int32)
    dst_fl = edge_index[1].astype(jnp.int32)
    ea_t_full = edge_attr.T
    offs = [0]
    for sz in SLAB_SIZES[:-1]:
        offs.append(offs[-1] + sz)

    src2, dst2, dstp = [], [], []
    for o, sz in zip(offs, SLAB_SIZES):
        ews = sz // NW
        src2.append(lax.slice_in_dim(src_fl, o, o + sz).reshape(NW, ews))
        d2 = lax.slice_in_dim(dst_fl, o, o + sz).reshape(NW, ews)
        dst2.append(d2)
        nchs = ews // ks
        pad = (nchs + 1) * ks - ews
        dstp.append(jnp.concatenate(
            [d2, jnp.full((NW, pad), NPAD - 1, jnp.int32)], axis=1
        ).reshape(NW, nchs + 1, ks))

    a, b, c = Wue1[:LAT], Wue1[LAT:2 * LAT], Wue1[2 * LAT:]
    u1a, u1b = Wun1[:LAT], Wun1[LAT:]
    ns = len(SLAB_SIZES)

    nodes, ps, pd = _node_encode(x, Wn1, Wn2, b, c)
    g1 = [_gather_pair(ps, pd, src2[t], dst2[t]) for t in range(ns)]
    e1, aggs = [], []
    for t in range(ns):
        e1.append(_edge_step1(ea_t_full, offs[t], g1[t], We1, We2, a, Wue2, blk))
        aggs.append(_segment_sum(e1[t], dstp[t], SLAB_SIZES[t] // NW))
    ps2, pd2 = _node_update(nodes, aggs, u1a, u1b, Wun2, b, c)
    dec, hs = [], []
    for t in range(ns):
        g2_t = _gather_pair(ps2, pd2, src2[t], dst2[t])
        dec_t, hp_t = _edge_step2(e1[t], g2_t, a, Wue2, Wd1, Wd2, blk)
        dec.append(dec_t)
        hs.append(jnp.sum(hp_t[:, 0, 0]))
    return jnp.concatenate(dec, axis=1).reshape(e, 1), sum(hs)


# speedup vs baseline: 5.3057x; 1.0150x over previous
import functools

import jax
import jax.numpy as jnp
from jax import lax
from jax.experimental import pallas as pl
from jax.experimental.pallas import tpu as pltpu
from jax.experimental.pallas import tpu_sc as plsc

LAT = 128
NCORES = 2
NSUB = 16
NW = NCORES * NSUB
CHUNK = 80

@functools.lru_cache(maxsize=1)
def _sc_mesh():
    return plsc.VectorSubcoreMesh(
        core_axis_name="c", subcore_axis_name="s",
        num_cores=NCORES, num_subcores=NSUB)


def _ln(h):
    mu = jnp.mean(h, axis=-1, keepdims=True)
    d = h - mu
    var = jnp.mean(d * d, axis=-1, keepdims=True)
    return d / jnp.sqrt(var + 1e-6)


def _relu(h):
    return jnp.maximum(h, 0.0)


def _node_encode_body(x_ref, wn1, wn2, b, c, nodes_ref, ps_ref, pd_ref):
    h = _relu(_relu(x_ref[...] @ wn1[...]) @ wn2[...])
    nodes = _ln(h)
    nodes_ref[...] = nodes
    ps_ref[...] = nodes @ b[...]
    pd_ref[...] = nodes @ c[...]


def _node_encode(x, wn1, wn2, b, c):
    n = x.shape[0]
    f = jax.ShapeDtypeStruct((n, LAT), jnp.float32)
    return pl.pallas_call(_node_encode_body, out_shape=(f, f, f))(x, wn1, wn2, b, c)


def _node_update_body(nodes_ref, *rest):
    aggs_refs = rest[:-7]
    u1a, u1b, wun2, b, c, ps_ref, pd_ref = rest[-7:]
    nodes = nodes_ref[...]
    n = nodes.shape[0]
    agg = sum(ar[i, :n, :] for ar in aggs_refs for i in range(NCORES))
    h = _relu(nodes @ u1a[...] + agg @ u1b[...])
    h = _relu(h @ wun2[...])
    n2 = nodes + _ln(h)
    ps_ref[...] = n2 @ b[...]
    pd_ref[...] = n2 @ c[...]


def _node_update(nodes, aggs, u1a, u1b, wun2, b, c):
    n = nodes.shape[0]
    f = jax.ShapeDtypeStruct((n, LAT), jnp.float32)
    return pl.pallas_call(_node_update_body, out_shape=(f, f))(
        nodes, *aggs, u1a, u1b, wun2, b, c)


def _bf(x):
    return x.astype(jnp.bfloat16)


def _mm(x, w):
    return lax.dot_general(_bf(x), _bf(w), (((1,), (0,)), ((), ())),
                           preferred_element_type=jnp.float32)


def _edge1_body(ea_ref, g_ref, we1, we2, a, wue2, e1_ref):
    h0 = lax.dot_general(_bf(ea_ref[...]), _bf(we1[...]),
                         (((0,), (0,)), ((), ())),
                         preferred_element_type=jnp.float32)
    e0 = _ln(_relu(_mm(_relu(h0), we2[...])))
    h = _relu(_mm(e0, a[...]) + g_ref[...])
    e1_ref[...] = e0 + _ln(_relu(_mm(h, wue2[...])))


def _edge_step1(ea_t_full, off, g, we1, we2, a, wue2, blk):
    de = ea_t_full.shape[0]
    e = g.shape[0]
    grid = (e // blk,)
    ob = off // blk
    full = lambda i: (0, 0)
    return pl.pallas_call(
        _edge1_body,
        grid=grid,
        in_specs=[
            pl.BlockSpec((de, blk), lambda i: (0, i + ob)),
            pl.BlockSpec((blk, LAT), lambda i: (i, 0)),
            pl.BlockSpec((de, LAT), full),
            pl.BlockSpec((LAT, LAT), full),
            pl.BlockSpec((LAT, LAT), full),
            pl.BlockSpec((LAT, LAT), full),
        ],
        out_specs=pl.BlockSpec((blk, LAT), lambda i: (i, 0)),
        out_shape=jax.ShapeDtypeStruct((e, LAT), jnp.float32),
        compiler_params=pltpu.CompilerParams(
            dimension_semantics=("arbitrary",)),
    )(ea_t_full, g, we1, we2, a, wue2)


def _edge2_body(e1_ref, g_ref, a, wue2, wd1, wd2, dec_ref, hp_ref):
    e1 = e1_ref[...]
    h = _relu(_mm(e1, a[...]) + g_ref[...])
    e2 = e1 + _ln(_relu(_mm(h, wue2[...])))
    h2 = _relu(_mm(e2, wd1[...]))
    d = lax.dot_general(_bf(wd2[...]), _bf(h2), (((0,), (1,)), ((), ())),
                        preferred_element_type=jnp.float32)
    dec_ref[...] = d
    hp_ref[...] = jnp.full((1, 1, LAT), jnp.sum(d), dtype=jnp.float32)


def _edge_step2(e1, g, a, wue2, wd1, wd2, blk):
    e = e1.shape[0]
    grid = (e // blk,)
    full = lambda i: (0, 0)
    return pl.pallas_call(
        _edge2_body,
        grid=grid,
        in_specs=[
            pl.BlockSpec((blk, LAT), lambda i: (i, 0)),
            pl.BlockSpec((blk, LAT), lambda i: (i, 0)),
            pl.BlockSpec((LAT, LAT), full),
            pl.BlockSpec((LAT, LAT), full),
            pl.BlockSpec((LAT, LAT), full),
            pl.BlockSpec((LAT, 1), full),
        ],
        out_specs=(
            pl.BlockSpec((1, blk), lambda i: (0, i)),
            pl.BlockSpec((1, 1, LAT), lambda i: (i, 0, 0)),
        ),
        out_shape=(
            jax.ShapeDtypeStruct((1, e), jnp.float32),
            jax.ShapeDtypeStruct((e // blk, 1, LAT), jnp.float32),
        ),
        compiler_params=pltpu.CompilerParams(
            dimension_semantics=("arbitrary",)),
    )(e1, g, a, wue2, wd1, wd2)


K2 = 128


def _gather_pair(ps, pd, src2, dst2):
    nw, ew = src2.shape
    e = nw * ew
    nch = ew // K2
    nchr = (nch // 2) * 2
    tail = ew - nch * K2

    @functools.partial(
        pl.kernel, mesh=_sc_mesh(),
        out_type=jax.ShapeDtypeStruct((e, LAT), jnp.float32),
        scratch_types=[
            pltpu.VMEM((ew,), jnp.int32),
            pltpu.VMEM((ew,), jnp.int32),
            pltpu.VMEM((2, K2, LAT), jnp.float32),
            pltpu.VMEM((2, K2, LAT), jnp.float32),
            pltpu.SemaphoreType.DMA,
            pltpu.SemaphoreType.DMA,
            pltpu.SemaphoreType.DMA,
            pltpu.SemaphoreType.DMA,
        ],
    )
    def kern(ps_hbm, pd_hbm, src_hbm, dst_hbm, out_hbm,
             idxs_v, idxd_v, bufp, bufd, g0, g1, w0, w1):
        wid = lax.axis_index("s") * NCORES + lax.axis_index("c")
        wbase = pl.multiple_of(wid * ew, 8)
        pltpu.sync_copy(src_hbm.at[wid], idxs_v)
        pltpu.sync_copy(dst_hbm.at[wid], idxd_v)
        gsem = (g0, g1)
        wsem = (w0, w1)

        def isl(v, i, k=K2):
            return v.at[pl.ds(pl.multiple_of(i * K2, 8), k)]

        def orows(i, k=K2):
            return out_hbm.at[pl.ds(pl.multiple_of(wbase + i * K2, 8), k)]

        def start_gather(i, b):
            pltpu.async_copy(ps_hbm.at[isl(idxs_v, i)], bufp.at[b], gsem[b])
            pltpu.async_copy(pd_hbm.at[isl(idxd_v, i)], bufd.at[b], gsem[b])

        def wait_gather(i, b):
            pltpu.make_async_copy(ps_hbm.at[isl(idxs_v, i)], bufp.at[b], gsem[b]).wait()
            pltpu.make_async_copy(pd_hbm.at[isl(idxd_v, i)], bufd.at[b], gsem[b]).wait()

        def add_buf(b, k):
            def addrow(r, carry):
                for j in range(LAT // 16):
                    sl = pl.ds(j * 16, 16)
                    bufp[b, r, sl] = bufp[b, r, sl] + bufd[b, r, sl]
                return carry
            lax.fori_loop(0, k, addrow, 0)

        start_gather(0, 0)
        start_gather(1, 1)

        def body(g, carry):
            for b in range(2):
                i = 2 * g + b
                wait_gather(i, b)
                add_buf(b, K2)
                pltpu.async_copy(bufp.at[b], orows(i), wsem[b])
            for b in range(2):
                i = 2 * g + b

                @pl.when(i + 2 < nchr)
                def _():
                    pltpu.make_async_copy(bufp.at[b], orows(i), wsem[b]).wait()
                    start_gather(i + 2, b)
            return carry
        lax.fori_loop(0, nchr // 2, body, 0)
        for b in range(2):
            pltpu.make_async_copy(bufp.at[b], orows(nchr - 2 + b), wsem[b]).wait()
        if nch > nchr:
            start_gather(nchr, 0)
            wait_gather(nchr, 0)
            add_buf(0, K2)
            pltpu.sync_copy(bufp.at[0], orows(nchr))
        if tail:
            cps = pltpu.async_copy(
                ps_hbm.at[isl(idxs_v, nch, tail)], bufp.at[0].at[pl.ds(0, tail)], g0)
            cpd = pltpu.async_copy(
                pd_hbm.at[isl(idxd_v, nch, tail)], bufd.at[0].at[pl.ds(0, tail)], g0)
            cps.wait()
            cpd.wait()
            add_buf(0, tail)
            pltpu.sync_copy(bufp.at[0].at[pl.ds(0, tail)], orows(nch, tail))

    return kern(ps, pd, src2, dst2)


def _segment_sum(e1, dstp, ew):
    nw, nchp, ks = dstp.shape
    nch = nchp - 1
    nchr = (nch // 2) * 2
    tail = ew - nch * ks
    npad = NPAD
    rpt = npad // NSUB
    stg = 64

    @functools.partial(
        pl.kernel, mesh=_sc_mesh(),
        out_type=jax.ShapeDtypeStruct((NCORES, npad, LAT), jnp.float32),
        scratch_types=[
            pltpu.VMEM((nchp, ks), jnp.int32),
            pltpu.VMEM((2, ks, LAT), jnp.float32),
            pltpu.VMEM((stg, LAT), jnp.float32),
            pltpu.VMEM_SHARED((npad, LAT), jnp.float32),
            pltpu.SemaphoreType.DMA,
            pltpu.SemaphoreType.DMA,
        ],
    )
    def kern(e_hbm, dst_hbm, out_hbm, idxd_v, rows, stage, acc_sh, r0, r1):
        c = lax.axis_index("c")
        s = lax.axis_index("s")
        wid = s * NCORES + c
        wbase = pl.multiple_of(wid * ew, 8)
        rsem = (r0, r1)

        def erows(i, k=ks):
            return e_hbm.at[pl.ds(pl.multiple_of(wbase + i * ks, 8), k)]

        pltpu.sync_copy(dst_hbm.at[wid], idxd_v)
        pltpu.async_copy(erows(0), rows.at[0], r0)
        pltpu.async_copy(erows(1), rows.at[1], r1)

        def zrows(ref, lo, cnt):
            def zrow(r, carry):
                for j in range(LAT // 16):
                    ref[r, pl.ds(j * 16, 16)] = jnp.zeros((16,), jnp.float32)
                return carry
            lax.fori_loop(lo, lo + cnt, zrow, 0)

        zrows(stage, 0, stg)
        for t in range(rpt // stg):
            pltpu.sync_copy(
                stage, acc_sh.at[pl.ds(pl.multiple_of(s * rpt + t * stg, 8), stg)])
        plsc.subcore_barrier()

        def body(g, carry):
            for b in range(2):
                i = 2 * g + b
                pltpu.make_async_copy(erows(i), rows.at[b], rsem[b]).wait()
                pltpu.sync_copy(rows.at[b], acc_sh.at[idxd_v.at[i]], add=True)

                @pl.when(i + 2 < nchr)
                def _():
                    pltpu.async_copy(erows(i + 2), rows.at[b], rsem[b])
            return carry
        lax.fori_loop(0, nchr // 2, body, 0)
        if nch > nchr:
            pltpu.async_copy(erows(nchr), rows.at[0], r0).wait()
            pltpu.sync_copy(rows.at[0], acc_sh.at[idxd_v.at[nchr]], add=True)
        pltpu.async_copy(erows(nch, tail), rows.at[0].at[pl.ds(0, tail)], r0).wait()
        def zrow2(r, carry):
            for j in range(LAT // 16):
                rows[0, r, pl.ds(j * 16, 16)] = jnp.zeros((16,), jnp.float32)
            return carry
        lax.fori_loop(tail, ks, zrow2, 0)
        pltpu.sync_copy(rows.at[0], acc_sh.at[idxd_v.at[nch]], add=True)
        plsc.subcore_barrier()

        for t in range(rpt // stg):
            sl = pl.ds(pl.multiple_of(s * rpt + t * stg, 8), stg)
            pltpu.sync_copy(acc_sh.at[sl], stage)
            pltpu.sync_copy(stage, out_hbm.at[c].at[sl])

    return kern(e1, dstp)


SLAB_SIZES = (76800, 102400, 140800)
NPAD = 10240


def kernel(x, edge_index, edge_attr, Wn1, Wn2, We1, We2,
           Wue1, Wue2, Wun1, Wun2, Wd1, Wd2):
    n = x.shape[0]
    e = edge_attr.shape[0]
    blk = 3200
    ks = 64

    src_fl = edge_index[0].astype(jnp.int32)
    dst_fl = edge_index[1].astype(jnp.int32)
    ea_t_full = edge_attr.T
    offs = [0]
    for sz in SLAB_SIZES[:-1]:
        offs.append(offs[-1] + sz)

    src2, dst2, dstp = [], [], []
    for o, sz in zip(offs, SLAB_SIZES):
        ews = sz // NW
        src2.append(lax.slice_in_dim(src_fl, o, o + sz).reshape(NW, ews))
        d2 = lax.slice_in_dim(dst_fl, o, o + sz).reshape(NW, ews)
        dst2.append(d2)
        nchs = ews // ks
        pad = (nchs + 1) * ks - ews
        dstp.append(jnp.concatenate(
            [d2, jnp.full((NW, pad), NPAD - 1, jnp.int32)], axis=1
        ).reshape(NW, nchs + 1, ks))

    a, b, c = Wue1[:LAT], Wue1[LAT:2 * LAT], Wue1[2 * LAT:]
    u1a, u1b = Wun1[:LAT], Wun1[LAT:]
    ns = len(SLAB_SIZES)

    nodes, ps, pd = _node_encode(x, Wn1, Wn2, b, c)
    g1 = [_gather_pair(ps, pd, src2[t], dst2[t]) for t in range(ns)]
    e1, aggs = [], []
    for t in range(ns):
        e1.append(_edge_step1(ea_t_full, offs[t], g1[t], We1, We2, a, Wue2, blk))
        aggs.append(_segment_sum(e1[t], dstp[t], SLAB_SIZES[t] // NW))
    ps2, pd2 = _node_update(nodes, aggs, u1a, u1b, Wun2, b, c)
    dec, hs = [], []
    for t in range(ns):
        g2_t = _gather_pair(ps2, pd2, src2[t], dst2[t])
        dec_t, hp_t = _edge_step2(e1[t], g2_t, a, Wue2, Wd1, Wd2, blk)
        dec.append(dec_t)
        hs.append(jnp.sum(hp_t[:, 0, 0]))
    return jnp.concatenate(dec, axis=1).reshape(e, 1), sum(hs)
